# baseline (speedup 1.0000x reference)
import functools

import jax
import jax.numpy as jnp
from jax import lax
from jax.experimental import pallas as pl
from jax.experimental.pallas import tpu as pltpu
from jax.experimental.pallas import tpu_sc as plsc

N = 10000
E = 320000
D = 128
H = 128
NB = 16
ETE = 16
L = 4

NCORE = 2
NSUB = 16
NW = NCORE * NSUB
CH = 128
K = 80
G = 40
EP = NW * K * CH
R2 = EP // CH
NP = 10240
RPS = NP // NSUB

RB = 512
TGRID = NP // RB
EB = 16

RK = 136
ABW = NP * RK
SLAB = ABW // NSUB
RPT = R2 // NSUB
BIG = 1e30


def _prep_body(x_ref, wp_ref, bp_ref, g_ref, b_ref, o_ref):
    z = jnp.dot(x_ref[...], wp_ref[...], preferred_element_type=jnp.float32)
    z = z + bp_ref[...]
    m = jnp.mean(z, axis=-1, keepdims=True)
    v = jnp.mean((z - m) ** 2, axis=-1, keepdims=True)
    z = (z - m) / jnp.sqrt(v + 1e-5) * g_ref[...] + b_ref[...]
    o_ref[...] = jnp.maximum(z, 0.0)


_prep = pl.pallas_call(
    _prep_body,
    grid=(TGRID,),
    in_specs=[
        pl.BlockSpec((RB, D), lambda i: (i, 0)),
        pl.BlockSpec((D, H), lambda i: (0, 0)),
        pl.BlockSpec((1, H), lambda i: (0, 0)),
        pl.BlockSpec((1, H), lambda i: (0, 0)),
        pl.BlockSpec((1, H), lambda i: (0, 0)),
    ],
    out_specs=pl.BlockSpec((RB, H), lambda i: (i, 0)),
    out_shape=jax.ShapeDtypeStruct((NP, H), jnp.float32),
)


def _cw_theta(emb_ref, we_ref, be_ref):
    c = jnp.dot(emb_ref[...], we_ref[:ETE, :],
                preferred_element_type=jnp.float32) + be_ref[...]
    w = we_ref[ETE, :][None, :]
    wz = w == 0.0
    th = -c / jnp.where(wz, 1.0, w)
    th = jnp.where(wz, BIG, th)
    th = jnp.clip(th, -BIG, BIG)
    return c, w, th


RKB = 4096


def _rank_body(dst_ref, ew_ref, emb_ref, we_ref, be_ref, o_ref, th_s):
    i = pl.program_id(0)

    @pl.when(i == 0)
    def _():
        _, _, th = _cw_theta(emb_ref, we_ref, be_ref)
        th_s[...] = th

    t = ew_ref[...]
    gt = (t > th_s[...]).astype(jnp.float32)
    r = jnp.sum(gt, axis=-1, keepdims=True).astype(jnp.int32)
    o_ref[...] = dst_ref[...] * RK + r


_rank = pl.pallas_call(
    _rank_body,
    grid=(EP // RKB,),
    in_specs=[
        pl.BlockSpec((RKB, 1), lambda i: (i, 0)),
        pl.BlockSpec((RKB, 1), lambda i: (i, 0)),
        pl.BlockSpec((1, ETE), lambda i: (0, 0)),
        pl.BlockSpec((ETE + 1, H), lambda i: (0, 0)),
        pl.BlockSpec((1, H), lambda i: (0, 0)),
    ],
    out_specs=pl.BlockSpec((RKB, 1), lambda i: (i, 0)),
    out_shape=jax.ShapeDtypeStruct((EP, 1), jnp.int32),
    scratch_shapes=[pltpu.VMEM((1, H), jnp.float32)],
)


def _etdeg_body(a_ref, b_ref, emb_ref, we_ref, be_ref, et_ref, deg_ref,
                tc_s, tw_s):
    i = pl.program_id(0)

    @pl.when(i == 0)
    def _():
        c, w, th = _cw_theta(emb_ref, we_ref, be_ref)
        thv = th[0]
        lt = (thv[:, None] < thv[None, :]).astype(jnp.float32)
        ik = lax.broadcasted_iota(jnp.int32, (H, H), 0)
        ih = lax.broadcasted_iota(jnp.int32, (H, H), 1)
        tie = ((thv[:, None] == thv[None, :]) & (ik < ih)).astype(jnp.float32)
        pos = jnp.sum(lt + tie, axis=0)
        rr = lax.broadcasted_iota(jnp.int32, (RK, H), 0)
        rr = rr.astype(jnp.float32)
        posb = pos[None, :]
        a1 = (rr > posb).astype(jnp.float32)
        m = ((w > 0.0).astype(jnp.float32) * a1
             + (w < 0.0).astype(jnp.float32) * (1.0 - a1)
             + (w == 0.0).astype(jnp.float32)
             * (c > 0.0).astype(jnp.float32))
        tc_s[...] = c * m
        tw_s[...] = w * m

    et_ref[...] = (
        jnp.dot(a_ref[...], tc_s[...], preferred_element_type=jnp.float32)
        + jnp.dot(b_ref[...], tw_s[...], preferred_element_type=jnp.float32))
    deg_ref[...] = jnp.sum(a_ref[...], axis=-1, keepdims=True)


_etdeg = pl.pallas_call(
    _etdeg_body,
    grid=(TGRID,),
    in_specs=[
        pl.BlockSpec((RB, RK), lambda i: (i, 0)),
        pl.BlockSpec((RB, RK), lambda i: (i, 0)),
        pl.BlockSpec((1, ETE), lambda i: (0, 0)),
        pl.BlockSpec((ETE + 1, H), lambda i: (0, 0)),
        pl.BlockSpec((1, H), lambda i: (0, 0)),
    ],
    out_specs=(
        pl.BlockSpec((RB, H), lambda i: (i, 0)),
        pl.BlockSpec((RB, 1), lambda i: (i, 0)),
    ),
    out_shape=(
        jax.ShapeDtypeStruct((NP, H), jnp.float32),
        jax.ShapeDtypeStruct((NP, 1), jnp.float32),
    ),
    scratch_shapes=[
        pltpu.VMEM((RK, H), jnp.float32),
        pltpu.VMEM((RK, H), jnp.float32),
    ],
)


@functools.cache
def _mesh():
    return plsc.VectorSubcoreMesh(
        core_axis_name="c", subcore_axis_name="s",
        num_cores=NCORE, num_subcores=NSUB)


def _ab_body(idx_hbm, ew_hbm, zer_hbm, ab_out,
             idx_v, val_v, ones_v, acc_sh, sem):
    c = lax.axis_index("c")
    s = lax.axis_index("s")
    pltpu.sync_copy(idx_hbm.at[pl.ds(s * RPT, RPT)], idx_v)

    @pl.when(c == 1)
    def _():
        pltpu.sync_copy(ew_hbm.at[pl.ds(s * RPT, RPT)], val_v)

    @pl.when(c == 0)
    def _():
        for i in range(CH // 16):
            ones_v[pl.ds(i * 16, 16)] = jnp.full((16,), 1.0, jnp.float32)

    pltpu.sync_copy(zer_hbm, acc_sh.at[pl.ds(s * SLAB, SLAB)])
    plsc.subcore_barrier()

    FD = 8

    @pl.loop(0, RPT, step=FD)
    def _chunks(j):
        for q in range(FD):
            @pl.when(c == 0)
            def _():
                pltpu.async_copy(ones_v, acc_sh.at[idx_v.at[j + q]], sem,
                                 add=True)

            @pl.when(c == 1)
            def _():
                pltpu.async_copy(val_v.at[j + q], acc_sh.at[idx_v.at[j + q]],
                                 sem, add=True)
        for q in range(FD):
            pltpu.make_async_copy(
                ones_v, acc_sh.at[idx_v.at[j + q]], sem).wait()

    plsc.subcore_barrier()
    pltpu.sync_copy(acc_sh.at[pl.ds(s * SLAB, SLAB)],
                    ab_out.at[c, pl.ds(s * SLAB, SLAB)])


@functools.cache
def _sc_ab():
    return pl.kernel(
        _ab_body,
        out_type=jax.ShapeDtypeStruct((NCORE, ABW), jnp.float32),
        mesh=_mesh(),
        scratch_types=[
            pltpu.VMEM((RPT, CH), jnp.int32),
            pltpu.VMEM((RPT, CH), jnp.float32),
            pltpu.VMEM((CH,), jnp.float32),
            pltpu.VMEM_SHARED((ABW,), jnp.float32),
            pltpu.SemaphoreType.DMA,
        ],
    )


def _agg_body(h_hbm, src_hbm, dst_hbm, zer_hbm, p_out,
              src_v, dst_v, buf0, buf1, p_sh, sem0, sem1):
    c = lax.axis_index("c")
    s = lax.axis_index("s")
    w = c * NSUB + s
    pltpu.sync_copy(zer_hbm, p_sh.at[pl.ds(s * RPS, RPS)])
    plsc.subcore_barrier()
    for half in range(K // G):
        pltpu.sync_copy(src_hbm.at[pl.ds(w * K + half * G, G)], src_v)
        pltpu.sync_copy(dst_hbm.at[pl.ds(w * K + half * G, G)], dst_v)
        pltpu.async_copy(h_hbm.at[src_v.at[0]], buf0, sem0)
        pltpu.async_copy(h_hbm.at[src_v.at[1]], buf1, sem1)

        @pl.loop(0, G, step=2)
        def _chunks(j):
            pltpu.make_async_copy(h_hbm.at[pl.ds(0, CH)], buf0, sem0).wait()
            pltpu.sync_copy(buf0, p_sh.at[dst_v.at[j]], add=True)

            @pl.when(j + 2 < G)
            def _():
                pltpu.async_copy(h_hbm.at[src_v.at[j + 2]], buf0, sem0)

            pltpu.make_async_copy(h_hbm.at[pl.ds(0, CH)], buf1, sem1).wait()
            pltpu.sync_copy(buf1, p_sh.at[dst_v.at[j + 1]], add=True)

            @pl.when(j + 3 < G)
            def _():
                pltpu.async_copy(h_hbm.at[src_v.at[j + 3]], buf1, sem1)

    plsc.subcore_barrier()
    pltpu.sync_copy(p_sh.at[pl.ds(s * RPS, RPS)],
                    p_out.at[c, pl.ds(s * RPS, RPS)])


@functools.cache
def _sc_agg():
    return pl.kernel(
        _agg_body,
        out_type=jax.ShapeDtypeStruct((NCORE, NP, H), jnp.float32),
        mesh=_mesh(),
        scratch_types=[
            pltpu.VMEM((G, CH), jnp.int32),
            pltpu.VMEM((G, CH), jnp.int32),
            pltpu.VMEM((CH, H), jnp.float32),
            pltpu.VMEM((CH, H), jnp.float32),
            pltpu.VMEM_SHARED((NP, H), jnp.float32),
            pltpu.SemaphoreType.DMA,
            pltpu.SemaphoreType.DMA,
        ],
    )


def _layer_body(p_ref, et_ref, deg_ref, h_ref, basis_ref, comp_ref,
                root_ref, cb_ref, g_ref, b_ref, o_ref, w_s):
    i = pl.program_id(0)

    @pl.when(i == 0)
    def _():
        w_s[...] = jnp.sum(comp_ref[...][:, :, None] * basis_ref[...], axis=0)

    r = 1.0 / jnp.maximum(deg_ref[...], 1.0)
    ps = p_ref[0] + p_ref[1]
    agg = jnp.dot(ps, w_s[...], preferred_element_type=jnp.float32) * r
    et = 0.1 * et_ref[...] * r
    h = h_ref[...]
    z = agg + jnp.dot(h, root_ref[...],
                      preferred_element_type=jnp.float32) + cb_ref[...] + et
    m = jnp.mean(z, axis=-1, keepdims=True)
    v = jnp.mean((z - m) ** 2, axis=-1, keepdims=True)
    z = (z - m) / jnp.sqrt(v + 1e-5) * g_ref[...] + b_ref[...]
    o_ref[...] = jnp.maximum(z, 0.0) + h


_layer = pl.pallas_call(
    _layer_body,
    grid=(TGRID,),
    in_specs=[
        pl.BlockSpec((NCORE, RB, H), lambda i: (0, i, 0)),
        pl.BlockSpec((RB, H), lambda i: (i, 0)),
        pl.BlockSpec((RB, 1), lambda i: (i, 0)),
        pl.BlockSpec((RB, H), lambda i: (i, 0)),
        pl.BlockSpec((NB, H, H), lambda i: (0, 0, 0)),
        pl.BlockSpec((NB, 1), lambda i: (0, 0)),
        pl.BlockSpec((H, H), lambda i: (0, 0)),
        pl.BlockSpec((1, H), lambda i: (0, 0)),
        pl.BlockSpec((1, H), lambda i: (0, 0)),
        pl.BlockSpec((1, H), lambda i: (0, 0)),
    ],
    out_specs=pl.BlockSpec((RB, H), lambda i: (i, 0)),
    out_shape=jax.ShapeDtypeStruct((NP, H), jnp.float32),
    scratch_shapes=[pltpu.VMEM((H, H), jnp.float32)],
)


def kernel(x, edge_index, edge_attr, Wp, bp, lnp_g, lnp_b, emb, We, be,
           basis, comp, root, conv_bias, ln_g, ln_b):
    src = edge_index[0]
    dst = edge_index[1]
    ew = edge_attr[:, 1]
    pad_e = EP - E
    ar = jnp.arange(pad_e, dtype=jnp.int32)
    src_p = jnp.concatenate([src, (ar * 997) % N])
    dst_p = jnp.concatenate([dst, N + (ar % (NP - N))])
    ew_p = jnp.concatenate([ew, jnp.zeros((pad_e,), jnp.float32)])
    src2 = src_p.reshape(R2, CH)
    dst2 = dst_p.reshape(R2, CH)
    ew2 = ew_p.reshape(R2, CH)
    x_p = jnp.pad(x, ((0, NP - N), (0, 0)))
    zer = jnp.zeros((RPS, H), jnp.float32)
    zer_ab = jnp.zeros((SLAB,), jnp.float32)

    idxa = _rank(dst_p[:, None], ew_p[:, None], emb, We,
                 be[None]).reshape(R2, CH)
    ab = _sc_ab()(idxa, ew2, zer_ab)
    h = _prep(x_p, Wp, bp[None], lnp_g[None], lnp_b[None])
    et, deg = _etdeg(ab[0].reshape(NP, RK), ab[1].reshape(NP, RK),
                     emb, We, be[None])
    for l in range(L):
        p = _sc_agg()(h, src2, dst2, zer)
        h = _layer(p, et, deg, h, basis[l], comp[l].reshape(NB, 1),
                   root[l], conv_bias[l][None], ln_g[l][None], ln_b[l][None])
    return h[:N]


# speedup vs baseline: 1.4081x; 1.4081x over previous
import functools

import jax
import jax.numpy as jnp
from jax import lax
from jax.experimental import pallas as pl
from jax.experimental.pallas import tpu as pltpu
from jax.experimental.pallas import tpu_sc as plsc

N = 10000
E = 320000
D = 128
H = 128
NB = 16
ETE = 16
L = 4

NCORE = 2
NSUB = 16
NW = NCORE * NSUB
CH = 128
K = 80
G = 40
EP = NW * K * CH
R2 = EP // CH
NP = 10240
RPS = NP // NSUB

RB = 512
TGRID = NP // RB
EB = 16

RK = 136
ABW = NP * RK
SLAB = ABW // NSUB
RPT = R2 // NSUB
BIG = 1e30


def _prep_body(x_ref, wp_ref, bp_ref, g_ref, b_ref, o_ref):
    z = jnp.dot(x_ref[...], wp_ref[...], preferred_element_type=jnp.float32)
    z = z + bp_ref[...]
    m = jnp.mean(z, axis=-1, keepdims=True)
    v = jnp.mean((z - m) ** 2, axis=-1, keepdims=True)
    z = (z - m) / jnp.sqrt(v + 1e-5) * g_ref[...] + b_ref[...]
    o_ref[...] = jnp.maximum(z, 0.0)


_prep = pl.pallas_call(
    _prep_body,
    grid=(TGRID,),
    in_specs=[
        pl.BlockSpec((RB, D), lambda i: (i, 0)),
        pl.BlockSpec((D, H), lambda i: (0, 0)),
        pl.BlockSpec((1, H), lambda i: (0, 0)),
        pl.BlockSpec((1, H), lambda i: (0, 0)),
        pl.BlockSpec((1, H), lambda i: (0, 0)),
    ],
    out_specs=pl.BlockSpec((RB, H), lambda i: (i, 0)),
    out_shape=jax.ShapeDtypeStruct((NP, H), jnp.float32),
)


def _cw_theta(emb_ref, we_ref, be_ref):
    c = jnp.dot(emb_ref[...], we_ref[:ETE, :],
                preferred_element_type=jnp.float32) + be_ref[...]
    w = we_ref[ETE, :][None, :]
    wz = w == 0.0
    th = -c / jnp.where(wz, 1.0, w)
    th = jnp.where(wz, BIG, th)
    th = jnp.clip(th, -BIG, BIG)
    return c, w, th


def _rank_body(dst_ref, ew_ref, emb_ref, we_ref, be_ref, o_ref, th_s):
    i = pl.program_id(0)

    @pl.when(i == 0)
    def _():
        _, _, th = _cw_theta(emb_ref, we_ref, be_ref)
        thb = jnp.broadcast_to(th, (H, H))
        eye = (lax.broadcasted_iota(jnp.int32, (H, H), 0)
               == lax.broadcasted_iota(jnp.int32, (H, H), 1))
        th_s[...] = jnp.sum(jnp.where(eye, thb, 0.0), axis=-1, keepdims=True)

    t = ew_ref[...]
    gt = (t[:, None, :] > th_s[...][None, :, :]).astype(jnp.float32)
    r = jnp.sum(gt, axis=1).astype(jnp.int32)
    o_ref[...] = dst_ref[...] * RK + r


_rank = pl.pallas_call(
    _rank_body,
    grid=(R2 // EB,),
    in_specs=[
        pl.BlockSpec((EB, CH), lambda i: (i, 0)),
        pl.BlockSpec((EB, CH), lambda i: (i, 0)),
        pl.BlockSpec((1, ETE), lambda i: (0, 0)),
        pl.BlockSpec((ETE + 1, H), lambda i: (0, 0)),
        pl.BlockSpec((1, H), lambda i: (0, 0)),
    ],
    out_specs=pl.BlockSpec((EB, CH), lambda i: (i, 0)),
    out_shape=jax.ShapeDtypeStruct((R2, CH), jnp.int32),
    scratch_shapes=[pltpu.VMEM((H, 1), jnp.float32)],
)


def _etdeg_body(a_ref, b_ref, emb_ref, we_ref, be_ref, et_ref, deg_ref,
                tc_s, tw_s):
    i = pl.program_id(0)

    @pl.when(i == 0)
    def _():
        c, w, th = _cw_theta(emb_ref, we_ref, be_ref)
        thv = th[0]
        lt = (thv[:, None] < thv[None, :]).astype(jnp.float32)
        ik = lax.broadcasted_iota(jnp.int32, (H, H), 0)
        ih = lax.broadcasted_iota(jnp.int32, (H, H), 1)
        tie = ((thv[:, None] == thv[None, :]) & (ik < ih)).astype(jnp.float32)
        pos = jnp.sum(lt + tie, axis=0)
        rr = lax.broadcasted_iota(jnp.int32, (RK, H), 0)
        rr = rr.astype(jnp.float32)
        posb = pos[None, :]
        a1 = (rr > posb).astype(jnp.float32)
        m = ((w > 0.0).astype(jnp.float32) * a1
             + (w < 0.0).astype(jnp.float32) * (1.0 - a1)
             + (w == 0.0).astype(jnp.float32)
             * (c > 0.0).astype(jnp.float32))
        tc_s[...] = c * m
        tw_s[...] = w * m

    et_ref[...] = (
        jnp.dot(a_ref[...], tc_s[...], preferred_element_type=jnp.float32)
        + jnp.dot(b_ref[...], tw_s[...], preferred_element_type=jnp.float32))
    deg_ref[...] = jnp.sum(a_ref[...], axis=-1, keepdims=True)


_etdeg = pl.pallas_call(
    _etdeg_body,
    grid=(TGRID,),
    in_specs=[
        pl.BlockSpec((RB, RK), lambda i: (i, 0)),
        pl.BlockSpec((RB, RK), lambda i: (i, 0)),
        pl.BlockSpec((1, ETE), lambda i: (0, 0)),
        pl.BlockSpec((ETE + 1, H), lambda i: (0, 0)),
        pl.BlockSpec((1, H), lambda i: (0, 0)),
    ],
    out_specs=(
        pl.BlockSpec((RB, H), lambda i: (i, 0)),
        pl.BlockSpec((RB, 1), lambda i: (i, 0)),
    ),
    out_shape=(
        jax.ShapeDtypeStruct((NP, H), jnp.float32),
        jax.ShapeDtypeStruct((NP, 1), jnp.float32),
    ),
    scratch_shapes=[
        pltpu.VMEM((RK, H), jnp.float32),
        pltpu.VMEM((RK, H), jnp.float32),
    ],
)


@functools.cache
def _mesh():
    return plsc.VectorSubcoreMesh(
        core_axis_name="c", subcore_axis_name="s",
        num_cores=NCORE, num_subcores=NSUB)


def _ab_body(idx_hbm, ew_hbm, zer_hbm, ab_out,
             idx_v, val_v, ones_v, acc_sh, sem):
    c = lax.axis_index("c")
    s = lax.axis_index("s")
    pltpu.sync_copy(idx_hbm.at[pl.ds(s * RPT, RPT)], idx_v)

    @pl.when(c == 1)
    def _():
        pltpu.sync_copy(ew_hbm.at[pl.ds(s * RPT, RPT)], val_v)

    @pl.when(c == 0)
    def _():
        for i in range(CH // 16):
            ones_v[pl.ds(i * 16, 16)] = jnp.full((16,), 1.0, jnp.float32)

    pltpu.sync_copy(zer_hbm, acc_sh.at[pl.ds(s * SLAB, SLAB)])
    plsc.subcore_barrier()

    FD = 8

    @pl.loop(0, RPT, step=FD)
    def _chunks(j):
        for q in range(FD):
            @pl.when(c == 0)
            def _():
                pltpu.async_copy(ones_v, acc_sh.at[idx_v.at[j + q]], sem,
                                 add=True)

            @pl.when(c == 1)
            def _():
                pltpu.async_copy(val_v.at[j + q], acc_sh.at[idx_v.at[j + q]],
                                 sem, add=True)
        for q in range(FD):
            pltpu.make_async_copy(
                ones_v, acc_sh.at[idx_v.at[j + q]], sem).wait()

    plsc.subcore_barrier()
    pltpu.sync_copy(acc_sh.at[pl.ds(s * SLAB, SLAB)],
                    ab_out.at[c, pl.ds(s * SLAB, SLAB)])


@functools.cache
def _sc_ab():
    return pl.kernel(
        _ab_body,
        out_type=jax.ShapeDtypeStruct((NCORE, ABW), jnp.float32),
        mesh=_mesh(),
        scratch_types=[
            pltpu.VMEM((RPT, CH), jnp.int32),
            pltpu.VMEM((RPT, CH), jnp.float32),
            pltpu.VMEM((CH,), jnp.float32),
            pltpu.VMEM_SHARED((ABW,), jnp.float32),
            pltpu.SemaphoreType.DMA,
        ],
    )


def _agg_body(h_hbm, src_hbm, dst_hbm, zer_hbm, p_out,
              src_v, dst_v, buf0, buf1, p_sh, sem0, sem1):
    c = lax.axis_index("c")
    s = lax.axis_index("s")
    w = c * NSUB + s
    pltpu.sync_copy(zer_hbm, p_sh.at[pl.ds(s * RPS, RPS)])
    plsc.subcore_barrier()
    for half in range(K // G):
        pltpu.sync_copy(src_hbm.at[pl.ds(w * K + half * G, G)], src_v)
        pltpu.sync_copy(dst_hbm.at[pl.ds(w * K + half * G, G)], dst_v)
        pltpu.async_copy(h_hbm.at[src_v.at[0]], buf0, sem0)
        pltpu.async_copy(h_hbm.at[src_v.at[1]], buf1, sem1)

        @pl.loop(0, G, step=2)
        def _chunks(j):
            pltpu.make_async_copy(h_hbm.at[pl.ds(0, CH)], buf0, sem0).wait()
            pltpu.sync_copy(buf0, p_sh.at[dst_v.at[j]], add=True)

            @pl.when(j + 2 < G)
            def _():
                pltpu.async_copy(h_hbm.at[src_v.at[j + 2]], buf0, sem0)

            pltpu.make_async_copy(h_hbm.at[pl.ds(0, CH)], buf1, sem1).wait()
            pltpu.sync_copy(buf1, p_sh.at[dst_v.at[j + 1]], add=True)

            @pl.when(j + 3 < G)
            def _():
                pltpu.async_copy(h_hbm.at[src_v.at[j + 3]], buf1, sem1)

    plsc.subcore_barrier()
    pltpu.sync_copy(p_sh.at[pl.ds(s * RPS, RPS)],
                    p_out.at[c, pl.ds(s * RPS, RPS)])


@functools.cache
def _sc_agg():
    return pl.kernel(
        _agg_body,
        out_type=jax.ShapeDtypeStruct((NCORE, NP, H), jnp.float32),
        mesh=_mesh(),
        scratch_types=[
            pltpu.VMEM((G, CH), jnp.int32),
            pltpu.VMEM((G, CH), jnp.int32),
            pltpu.VMEM((CH, H), jnp.float32),
            pltpu.VMEM((CH, H), jnp.float32),
            pltpu.VMEM_SHARED((NP, H), jnp.float32),
            pltpu.SemaphoreType.DMA,
            pltpu.SemaphoreType.DMA,
        ],
    )


def _layer_body(p_ref, et_ref, deg_ref, h_ref, basis_ref, comp_ref,
                root_ref, cb_ref, g_ref, b_ref, o_ref, w_s):
    i = pl.program_id(0)

    @pl.when(i == 0)
    def _():
        w_s[...] = jnp.sum(comp_ref[...][:, :, None] * basis_ref[...], axis=0)

    r = 1.0 / jnp.maximum(deg_ref[...], 1.0)
    ps = p_ref[0] + p_ref[1]
    agg = jnp.dot(ps, w_s[...], preferred_element_type=jnp.float32) * r
    et = 0.1 * et_ref[...] * r
    h = h_ref[...]
    z = agg + jnp.dot(h, root_ref[...],
                      preferred_element_type=jnp.float32) + cb_ref[...] + et
    m = jnp.mean(z, axis=-1, keepdims=True)
    v = jnp.mean((z - m) ** 2, axis=-1, keepdims=True)
    z = (z - m) / jnp.sqrt(v + 1e-5) * g_ref[...] + b_ref[...]
    o_ref[...] = jnp.maximum(z, 0.0) + h


_layer = pl.pallas_call(
    _layer_body,
    grid=(TGRID,),
    in_specs=[
        pl.BlockSpec((NCORE, RB, H), lambda i: (0, i, 0)),
        pl.BlockSpec((RB, H), lambda i: (i, 0)),
        pl.BlockSpec((RB, 1), lambda i: (i, 0)),
        pl.BlockSpec((RB, H), lambda i: (i, 0)),
        pl.BlockSpec((NB, H, H), lambda i: (0, 0, 0)),
        pl.BlockSpec((NB, 1), lambda i: (0, 0)),
        pl.BlockSpec((H, H), lambda i: (0, 0)),
        pl.BlockSpec((1, H), lambda i: (0, 0)),
        pl.BlockSpec((1, H), lambda i: (0, 0)),
        pl.BlockSpec((1, H), lambda i: (0, 0)),
    ],
    out_specs=pl.BlockSpec((RB, H), lambda i: (i, 0)),
    out_shape=jax.ShapeDtypeStruct((NP, H), jnp.float32),
    scratch_shapes=[pltpu.VMEM((H, H), jnp.float32)],
)


def kernel(x, edge_index, edge_attr, Wp, bp, lnp_g, lnp_b, emb, We, be,
           basis, comp, root, conv_bias, ln_g, ln_b):
    src = edge_index[0]
    dst = edge_index[1]
    ew = edge_attr[:, 1]
    pad_e = EP - E
    ar = jnp.arange(pad_e, dtype=jnp.int32)
    src_p = jnp.concatenate([src, (ar * 997) % N])
    dst_p = jnp.concatenate([dst, N + (ar % (NP - N))])
    ew_p = jnp.concatenate([ew, jnp.zeros((pad_e,), jnp.float32)])
    src2 = src_p.reshape(R2, CH)
    dst2 = dst_p.reshape(R2, CH)
    ew2 = ew_p.reshape(R2, CH)
    x_p = jnp.pad(x, ((0, NP - N), (0, 0)))
    zer = jnp.zeros((RPS, H), jnp.float32)
    zer_ab = jnp.zeros((SLAB,), jnp.float32)

    idxa = _rank(dst2, ew2, emb, We, be[None])
    ab = _sc_ab()(idxa, ew2, zer_ab)
    h = _prep(x_p, Wp, bp[None], lnp_g[None], lnp_b[None])
    et, deg = _etdeg(ab[0].reshape(NP, RK), ab[1].reshape(NP, RK),
                     emb, We, be[None])
    for l in range(L):
        p = _sc_agg()(h, src2, dst2, zer)
        h = _layer(p, et, deg, h, basis[l], comp[l].reshape(NB, 1),
                   root[l], conv_bias[l][None], ln_g[l][None], ln_b[l][None])
    return h[:N]


# speedup vs baseline: 1.4878x; 1.0566x over previous
import functools

import jax
import jax.numpy as jnp
from jax import lax
from jax.experimental import pallas as pl
from jax.experimental.pallas import tpu as pltpu
from jax.experimental.pallas import tpu_sc as plsc

N = 10000
E = 320000
D = 128
H = 128
NB = 16
ETE = 16
L = 4

NCORE = 2
NSUB = 16
NW = NCORE * NSUB
CH = 128
K = 80
G = 40
EP = NW * K * CH
R2 = EP // CH
NP = 10240
RPS = NP // NSUB

RB = 512
TGRID = NP // RB
EB = 32

RK = 136
ABW = NP * RK
SLAB = ABW // NSUB
RPT = R2 // NSUB
BIG = 1e30


def _prep_body(x_ref, wp_ref, bp_ref, g_ref, b_ref, o_ref):
    z = jnp.dot(x_ref[...], wp_ref[...], preferred_element_type=jnp.float32)
    z = z + bp_ref[...]
    m = jnp.mean(z, axis=-1, keepdims=True)
    v = jnp.mean((z - m) ** 2, axis=-1, keepdims=True)
    z = (z - m) / jnp.sqrt(v + 1e-5) * g_ref[...] + b_ref[...]
    o_ref[...] = jnp.maximum(z, 0.0)


_prep = pl.pallas_call(
    _prep_body,
    grid=(TGRID,),
    in_specs=[
        pl.BlockSpec((RB, D), lambda i: (i, 0)),
        pl.BlockSpec((D, H), lambda i: (0, 0)),
        pl.BlockSpec((1, H), lambda i: (0, 0)),
        pl.BlockSpec((1, H), lambda i: (0, 0)),
        pl.BlockSpec((1, H), lambda i: (0, 0)),
    ],
    out_specs=pl.BlockSpec((RB, H), lambda i: (i, 0)),
    out_shape=jax.ShapeDtypeStruct((NP, H), jnp.float32),
)


def _cw_theta(emb_ref, we_ref, be_ref):
    c = jnp.dot(emb_ref[...], we_ref[:ETE, :],
                preferred_element_type=jnp.float32) + be_ref[...]
    w = we_ref[ETE, :][None, :]
    wz = w == 0.0
    th = -c / jnp.where(wz, 1.0, w)
    th = jnp.where(wz, BIG, th)
    th = jnp.clip(th, -BIG, BIG)
    return c, w, th


def _rank_body(dst_ref, ew_ref, emb_ref, we_ref, be_ref, o_ref, th_s):
    i = pl.program_id(0)

    @pl.when(i == 0)
    def _():
        _, _, th = _cw_theta(emb_ref, we_ref, be_ref)
        thb = jnp.broadcast_to(th, (H, H))
        eye = (lax.broadcasted_iota(jnp.int32, (H, H), 0)
               == lax.broadcasted_iota(jnp.int32, (H, H), 1))
        th_s[...] = jnp.sum(jnp.where(eye, thb, 0.0), axis=-1, keepdims=True)

    t = ew_ref[...]
    gt = (t[:, None, :] > th_s[...][None, :, :]).astype(jnp.float32)
    r = jnp.sum(gt, axis=1).astype(jnp.int32)
    o_ref[...] = dst_ref[...] * RK + r


_rank = pl.pallas_call(
    _rank_body,
    grid=(R2 // EB,),
    in_specs=[
        pl.BlockSpec((EB, CH), lambda i: (i, 0)),
        pl.BlockSpec((EB, CH), lambda i: (i, 0)),
        pl.BlockSpec((1, ETE), lambda i: (0, 0)),
        pl.BlockSpec((ETE + 1, H), lambda i: (0, 0)),
        pl.BlockSpec((1, H), lambda i: (0, 0)),
    ],
    out_specs=pl.BlockSpec((EB, CH), lambda i: (i, 0)),
    out_shape=jax.ShapeDtypeStruct((R2, CH), jnp.int32),
    scratch_shapes=[pltpu.VMEM((H, 1), jnp.float32)],
)


def _etdeg_body(a_ref, b_ref, emb_ref, we_ref, be_ref, et_ref, deg_ref,
                tc_s, tw_s):
    i = pl.program_id(0)

    @pl.when(i == 0)
    def _():
        c, w, th = _cw_theta(emb_ref, we_ref, be_ref)
        thv = th[0]
        lt = (thv[:, None] < thv[None, :]).astype(jnp.float32)
        ik = lax.broadcasted_iota(jnp.int32, (H, H), 0)
        ih = lax.broadcasted_iota(jnp.int32, (H, H), 1)
        tie = ((thv[:, None] == thv[None, :]) & (ik < ih)).astype(jnp.float32)
        pos = jnp.sum(lt + tie, axis=0)
        rr = lax.broadcasted_iota(jnp.int32, (RK, H), 0)
        rr = rr.astype(jnp.float32)
        posb = pos[None, :]
        a1 = (rr > posb).astype(jnp.float32)
        m = ((w > 0.0).astype(jnp.float32) * a1
             + (w < 0.0).astype(jnp.float32) * (1.0 - a1)
             + (w == 0.0).astype(jnp.float32)
             * (c > 0.0).astype(jnp.float32))
        tc_s[...] = c * m
        tw_s[...] = w * m

    et_ref[...] = (
        jnp.dot(a_ref[...], tc_s[...], preferred_element_type=jnp.float32)
        + jnp.dot(b_ref[...], tw_s[...], preferred_element_type=jnp.float32))
    deg_ref[...] = jnp.sum(a_ref[...], axis=-1, keepdims=True)


_etdeg = pl.pallas_call(
    _etdeg_body,
    grid=(TGRID,),
    in_specs=[
        pl.BlockSpec((RB, RK), lambda i: (i, 0)),
        pl.BlockSpec((RB, RK), lambda i: (i, 0)),
        pl.BlockSpec((1, ETE), lambda i: (0, 0)),
        pl.BlockSpec((ETE + 1, H), lambda i: (0, 0)),
        pl.BlockSpec((1, H), lambda i: (0, 0)),
    ],
    out_specs=(
        pl.BlockSpec((RB, H), lambda i: (i, 0)),
        pl.BlockSpec((RB, 1), lambda i: (i, 0)),
    ),
    out_shape=(
        jax.ShapeDtypeStruct((NP, H), jnp.float32),
        jax.ShapeDtypeStruct((NP, 1), jnp.float32),
    ),
    scratch_shapes=[
        pltpu.VMEM((RK, H), jnp.float32),
        pltpu.VMEM((RK, H), jnp.float32),
    ],
)


@functools.cache
def _mesh():
    return plsc.VectorSubcoreMesh(
        core_axis_name="c", subcore_axis_name="s",
        num_cores=NCORE, num_subcores=NSUB)


def _ab_body(idx_hbm, ew_hbm, zer_hbm, ab_out,
             idx_v, val_v, ones_v, acc_sh, sem):
    c = lax.axis_index("c")
    s = lax.axis_index("s")
    pltpu.sync_copy(idx_hbm.at[pl.ds(s * RPT, RPT)], idx_v)

    @pl.when(c == 1)
    def _():
        pltpu.sync_copy(ew_hbm.at[pl.ds(s * RPT, RPT)], val_v)

    @pl.when(c == 0)
    def _():
        for i in range(CH // 16):
            ones_v[pl.ds(i * 16, 16)] = jnp.full((16,), 1.0, jnp.float32)

    pltpu.sync_copy(zer_hbm, acc_sh.at[pl.ds(s * SLAB, SLAB)])
    plsc.subcore_barrier()

    FD = 8

    @pl.loop(0, RPT, step=FD)
    def _chunks(j):
        for q in range(FD):
            @pl.when(c == 0)
            def _():
                pltpu.async_copy(ones_v, acc_sh.at[idx_v.at[j + q]], sem,
                                 add=True)

            @pl.when(c == 1)
            def _():
                pltpu.async_copy(val_v.at[j + q], acc_sh.at[idx_v.at[j + q]],
                                 sem, add=True)
        for q in range(FD):
            pltpu.make_async_copy(
                ones_v, acc_sh.at[idx_v.at[j + q]], sem).wait()

    plsc.subcore_barrier()
    pltpu.sync_copy(acc_sh.at[pl.ds(s * SLAB, SLAB)],
                    ab_out.at[c, pl.ds(s * SLAB, SLAB)])


@functools.cache
def _sc_ab():
    return pl.kernel(
        _ab_body,
        out_type=jax.ShapeDtypeStruct((NCORE, ABW), jnp.float32),
        mesh=_mesh(),
        scratch_types=[
            pltpu.VMEM((RPT, CH), jnp.int32),
            pltpu.VMEM((RPT, CH), jnp.float32),
            pltpu.VMEM((CH,), jnp.float32),
            pltpu.VMEM_SHARED((ABW,), jnp.float32),
            pltpu.SemaphoreType.DMA,
        ],
    )


def _agg_body(h_hbm, src_hbm, dst_hbm, zer_hbm, p_out,
              src_v, dst_v, buf0, buf1, p_sh, sem0, sem1):
    c = lax.axis_index("c")
    s = lax.axis_index("s")
    w = c * NSUB + s
    pltpu.sync_copy(zer_hbm, p_sh.at[pl.ds(s * RPS, RPS)])
    plsc.subcore_barrier()
    for half in range(K // G):
        pltpu.sync_copy(src_hbm.at[pl.ds(w * K + half * G, G)], src_v)
        pltpu.sync_copy(dst_hbm.at[pl.ds(w * K + half * G, G)], dst_v)
        pltpu.async_copy(h_hbm.at[src_v.at[0]], buf0, sem0)
        pltpu.async_copy(h_hbm.at[src_v.at[1]], buf1, sem1)

        @pl.loop(0, G, step=2)
        def _chunks(j):
            pltpu.make_async_copy(h_hbm.at[pl.ds(0, CH)], buf0, sem0).wait()
            pltpu.sync_copy(buf0, p_sh.at[dst_v.at[j]], add=True)

            @pl.when(j + 2 < G)
            def _():
                pltpu.async_copy(h_hbm.at[src_v.at[j + 2]], buf0, sem0)

            pltpu.make_async_copy(h_hbm.at[pl.ds(0, CH)], buf1, sem1).wait()
            pltpu.sync_copy(buf1, p_sh.at[dst_v.at[j + 1]], add=True)

            @pl.when(j + 3 < G)
            def _():
                pltpu.async_copy(h_hbm.at[src_v.at[j + 3]], buf1, sem1)

    plsc.subcore_barrier()
    pltpu.sync_copy(p_sh.at[pl.ds(s * RPS, RPS)],
                    p_out.at[c, pl.ds(s * RPS, RPS)])


@functools.cache
def _sc_agg():
    return pl.kernel(
        _agg_body,
        out_type=jax.ShapeDtypeStruct((NCORE, NP, H), jnp.float32),
        mesh=_mesh(),
        scratch_types=[
            pltpu.VMEM((G, CH), jnp.int32),
            pltpu.VMEM((G, CH), jnp.int32),
            pltpu.VMEM((CH, H), jnp.float32),
            pltpu.VMEM((CH, H), jnp.float32),
            pltpu.VMEM_SHARED((NP, H), jnp.float32),
            pltpu.SemaphoreType.DMA,
            pltpu.SemaphoreType.DMA,
        ],
    )


def _layer_body(p_ref, et_ref, deg_ref, h_ref, basis_ref, comp_ref,
                root_ref, cb_ref, g_ref, b_ref, o_ref, w_s):
    i = pl.program_id(0)

    @pl.when(i == 0)
    def _():
        w_s[...] = jnp.sum(comp_ref[...][:, :, None] * basis_ref[...], axis=0)

    r = 1.0 / jnp.maximum(deg_ref[...], 1.0)
    ps = p_ref[0] + p_ref[1]
    agg = jnp.dot(ps, w_s[...], preferred_element_type=jnp.float32) * r
    et = 0.1 * et_ref[...] * r
    h = h_ref[...]
    z = agg + jnp.dot(h, root_ref[...],
                      preferred_element_type=jnp.float32) + cb_ref[...] + et
    m = jnp.mean(z, axis=-1, keepdims=True)
    v = jnp.mean((z - m) ** 2, axis=-1, keepdims=True)
    z = (z - m) / jnp.sqrt(v + 1e-5) * g_ref[...] + b_ref[...]
    o_ref[...] = jnp.maximum(z, 0.0) + h


_layer = pl.pallas_call(
    _layer_body,
    grid=(TGRID,),
    in_specs=[
        pl.BlockSpec((NCORE, RB, H), lambda i: (0, i, 0)),
        pl.BlockSpec((RB, H), lambda i: (i, 0)),
        pl.BlockSpec((RB, 1), lambda i: (i, 0)),
        pl.BlockSpec((RB, H), lambda i: (i, 0)),
        pl.BlockSpec((NB, H, H), lambda i: (0, 0, 0)),
        pl.BlockSpec((NB, 1), lambda i: (0, 0)),
        pl.BlockSpec((H, H), lambda i: (0, 0)),
        pl.BlockSpec((1, H), lambda i: (0, 0)),
        pl.BlockSpec((1, H), lambda i: (0, 0)),
        pl.BlockSpec((1, H), lambda i: (0, 0)),
    ],
    out_specs=pl.BlockSpec((RB, H), lambda i: (i, 0)),
    out_shape=jax.ShapeDtypeStruct((NP, H), jnp.float32),
    scratch_shapes=[pltpu.VMEM((H, H), jnp.float32)],
)


def kernel(x, edge_index, edge_attr, Wp, bp, lnp_g, lnp_b, emb, We, be,
           basis, comp, root, conv_bias, ln_g, ln_b):
    src = edge_index[0]
    dst = edge_index[1]
    ew = edge_attr[:, 1]
    pad_e = EP - E
    ar = jnp.arange(pad_e, dtype=jnp.int32)
    src_p = jnp.concatenate([src, (ar * 997) % N])
    dst_p = jnp.concatenate([dst, N + (ar % (NP - N))])
    ew_p = jnp.concatenate([ew, jnp.zeros((pad_e,), jnp.float32)])
    src2 = src_p.reshape(R2, CH)
    dst2 = dst_p.reshape(R2, CH)
    ew2 = ew_p.reshape(R2, CH)
    x_p = jnp.pad(x, ((0, NP - N), (0, 0)))
    zer = jnp.zeros((RPS, H), jnp.float32)
    zer_ab = jnp.zeros((SLAB,), jnp.float32)

    idxa = _rank(dst2, ew2, emb, We, be[None])
    ab = _sc_ab()(idxa, ew2, zer_ab)
    h = _prep(x_p, Wp, bp[None], lnp_g[None], lnp_b[None])
    et, deg = _etdeg(ab[0].reshape(NP, RK), ab[1].reshape(NP, RK),
                     emb, We, be[None])
    for l in range(L):
        p = _sc_agg()(h, src2, dst2, zer)
        h = _layer(p, et, deg, h, basis[l], comp[l].reshape(NB, 1),
                   root[l], conv_bias[l][None], ln_g[l][None], ln_b[l][None])
    return h[:N]


# speedup vs baseline: 1.5426x; 1.0369x over previous
import functools

import jax
import jax.numpy as jnp
from jax import lax
from jax.experimental import pallas as pl
from jax.experimental.pallas import tpu as pltpu
from jax.experimental.pallas import tpu_sc as plsc

N = 10000
E = 320000
D = 128
H = 128
NB = 16
ETE = 16
L = 4

NCORE = 2
NSUB = 16
NW = NCORE * NSUB
CH = 128
K = 80
G = 40
EP = NW * K * CH
R2 = EP // CH
NP = 10240
RPS = NP // NSUB

RB = 512
TGRID = NP // RB
EB = 128

RK = 136
ABW = NP * RK
SLAB = ABW // NSUB
RPT = R2 // NSUB
BIG = 1e30


def _prep_body(x_ref, wp_ref, bp_ref, g_ref, b_ref, o_ref):
    z = jnp.dot(x_ref[...], wp_ref[...], preferred_element_type=jnp.float32)
    z = z + bp_ref[...]
    m = jnp.mean(z, axis=-1, keepdims=True)
    v = jnp.mean((z - m) ** 2, axis=-1, keepdims=True)
    z = (z - m) / jnp.sqrt(v + 1e-5) * g_ref[...] + b_ref[...]
    o_ref[...] = jnp.maximum(z, 0.0)


_prep = pl.pallas_call(
    _prep_body,
    grid=(TGRID,),
    in_specs=[
        pl.BlockSpec((RB, D), lambda i: (i, 0)),
        pl.BlockSpec((D, H), lambda i: (0, 0)),
        pl.BlockSpec((1, H), lambda i: (0, 0)),
        pl.BlockSpec((1, H), lambda i: (0, 0)),
        pl.BlockSpec((1, H), lambda i: (0, 0)),
    ],
    out_specs=pl.BlockSpec((RB, H), lambda i: (i, 0)),
    out_shape=jax.ShapeDtypeStruct((NP, H), jnp.float32),
)


def _cw_theta(emb_ref, we_ref, be_ref):
    c = jnp.dot(emb_ref[...], we_ref[:ETE, :],
                preferred_element_type=jnp.float32) + be_ref[...]
    w = we_ref[ETE, :][None, :]
    wz = w == 0.0
    th = -c / jnp.where(wz, 1.0, w)
    th = jnp.where(wz, BIG, th)
    th = jnp.clip(th, -BIG, BIG)
    return c, w, th


def _rank_body(dst_ref, ew_ref, emb_ref, we_ref, be_ref, o_ref, th_s):
    i = pl.program_id(0)

    @pl.when(i == 0)
    def _():
        _, _, th = _cw_theta(emb_ref, we_ref, be_ref)
        thb = jnp.broadcast_to(th, (H, H))
        eye = (lax.broadcasted_iota(jnp.int32, (H, H), 0)
               == lax.broadcasted_iota(jnp.int32, (H, H), 1))
        th_s[...] = jnp.sum(jnp.where(eye, thb, 0.0), axis=-1, keepdims=True)

    t = ew_ref[...]
    gt = (t[:, None, :] > th_s[...][None, :, :]).astype(jnp.float32)
    r = jnp.sum(gt, axis=1).astype(jnp.int32)
    o_ref[...] = dst_ref[...] * RK + r


_rank = pl.pallas_call(
    _rank_body,
    grid=(R2 // EB,),
    in_specs=[
        pl.BlockSpec((EB, CH), lambda i: (i, 0)),
        pl.BlockSpec((EB, CH), lambda i: (i, 0)),
        pl.BlockSpec((1, ETE), lambda i: (0, 0)),
        pl.BlockSpec((ETE + 1, H), lambda i: (0, 0)),
        pl.BlockSpec((1, H), lambda i: (0, 0)),
    ],
    out_specs=pl.BlockSpec((EB, CH), lambda i: (i, 0)),
    out_shape=jax.ShapeDtypeStruct((R2, CH), jnp.int32),
    scratch_shapes=[pltpu.VMEM((H, 1), jnp.float32)],
)


def _etdeg_body(a_ref, b_ref, emb_ref, we_ref, be_ref, et_ref, deg_ref,
                tc_s, tw_s):
    i = pl.program_id(0)

    @pl.when(i == 0)
    def _():
        c, w, th = _cw_theta(emb_ref, we_ref, be_ref)
        thv = th[0]
        lt = (thv[:, None] < thv[None, :]).astype(jnp.float32)
        ik = lax.broadcasted_iota(jnp.int32, (H, H), 0)
        ih = lax.broadcasted_iota(jnp.int32, (H, H), 1)
        tie = ((thv[:, None] == thv[None, :]) & (ik < ih)).astype(jnp.float32)
        pos = jnp.sum(lt + tie, axis=0)
        rr = lax.broadcasted_iota(jnp.int32, (RK, H), 0)
        rr = rr.astype(jnp.float32)
        posb = pos[None, :]
        a1 = (rr > posb).astype(jnp.float32)
        m = ((w > 0.0).astype(jnp.float32) * a1
             + (w < 0.0).astype(jnp.float32) * (1.0 - a1)
             + (w == 0.0).astype(jnp.float32)
             * (c > 0.0).astype(jnp.float32))
        tc_s[...] = c * m
        tw_s[...] = w * m

    et_ref[...] = (
        jnp.dot(a_ref[...], tc_s[...], preferred_element_type=jnp.float32)
        + jnp.dot(b_ref[...], tw_s[...], preferred_element_type=jnp.float32))
    deg_ref[...] = jnp.sum(a_ref[...], axis=-1, keepdims=True)


_etdeg = pl.pallas_call(
    _etdeg_body,
    grid=(TGRID,),
    in_specs=[
        pl.BlockSpec((RB, RK), lambda i: (i, 0)),
        pl.BlockSpec((RB, RK), lambda i: (i, 0)),
        pl.BlockSpec((1, ETE), lambda i: (0, 0)),
        pl.BlockSpec((ETE + 1, H), lambda i: (0, 0)),
        pl.BlockSpec((1, H), lambda i: (0, 0)),
    ],
    out_specs=(
        pl.BlockSpec((RB, H), lambda i: (i, 0)),
        pl.BlockSpec((RB, 1), lambda i: (i, 0)),
    ),
    out_shape=(
        jax.ShapeDtypeStruct((NP, H), jnp.float32),
        jax.ShapeDtypeStruct((NP, 1), jnp.float32),
    ),
    scratch_shapes=[
        pltpu.VMEM((RK, H), jnp.float32),
        pltpu.VMEM((RK, H), jnp.float32),
    ],
)


@functools.cache
def _mesh():
    return plsc.VectorSubcoreMesh(
        core_axis_name="c", subcore_axis_name="s",
        num_cores=NCORE, num_subcores=NSUB)


def _ab_body(idx_hbm, ew_hbm, zer_hbm, ab_out,
             idx_v, val_v, ones_v, acc_sh, sem):
    c = lax.axis_index("c")
    s = lax.axis_index("s")
    pltpu.sync_copy(idx_hbm.at[pl.ds(s * RPT, RPT)], idx_v)

    @pl.when(c == 1)
    def _():
        pltpu.sync_copy(ew_hbm.at[pl.ds(s * RPT, RPT)], val_v)

    @pl.when(c == 0)
    def _():
        for i in range(CH // 16):
            ones_v[pl.ds(i * 16, 16)] = jnp.full((16,), 1.0, jnp.float32)

    pltpu.sync_copy(zer_hbm, acc_sh.at[pl.ds(s * SLAB, SLAB)])
    plsc.subcore_barrier()

    FD = 8

    @pl.loop(0, RPT, step=FD)
    def _chunks(j):
        for q in range(FD):
            @pl.when(c == 0)
            def _():
                pltpu.async_copy(ones_v, acc_sh.at[idx_v.at[j + q]], sem,
                                 add=True)

            @pl.when(c == 1)
            def _():
                pltpu.async_copy(val_v.at[j + q], acc_sh.at[idx_v.at[j + q]],
                                 sem, add=True)
        for q in range(FD):
            pltpu.make_async_copy(
                ones_v, acc_sh.at[idx_v.at[j + q]], sem).wait()

    plsc.subcore_barrier()
    pltpu.sync_copy(acc_sh.at[pl.ds(s * SLAB, SLAB)],
                    ab_out.at[c, pl.ds(s * SLAB, SLAB)])


@functools.cache
def _sc_ab():
    return pl.kernel(
        _ab_body,
        out_type=jax.ShapeDtypeStruct((NCORE, ABW), jnp.float32),
        mesh=_mesh(),
        scratch_types=[
            pltpu.VMEM((RPT, CH), jnp.int32),
            pltpu.VMEM((RPT, CH), jnp.float32),
            pltpu.VMEM((CH,), jnp.float32),
            pltpu.VMEM_SHARED((ABW,), jnp.float32),
            pltpu.SemaphoreType.DMA,
        ],
    )


def _agg_body(h_hbm, src_hbm, dst_hbm, zer_hbm, p_out,
              src_v, dst_v, buf0, buf1, p_sh, sem0, sem1):
    c = lax.axis_index("c")
    s = lax.axis_index("s")
    w = c * NSUB + s
    pltpu.sync_copy(zer_hbm, p_sh.at[pl.ds(s * RPS, RPS)])
    plsc.subcore_barrier()
    for half in range(K // G):
        pltpu.sync_copy(src_hbm.at[pl.ds(w * K + half * G, G)], src_v)
        pltpu.sync_copy(dst_hbm.at[pl.ds(w * K + half * G, G)], dst_v)
        pltpu.async_copy(h_hbm.at[src_v.at[0]], buf0, sem0)
        pltpu.async_copy(h_hbm.at[src_v.at[1]], buf1, sem1)

        @pl.loop(0, G, step=2)
        def _chunks(j):
            pltpu.make_async_copy(h_hbm.at[pl.ds(0, CH)], buf0, sem0).wait()
            pltpu.sync_copy(buf0, p_sh.at[dst_v.at[j]], add=True)

            @pl.when(j + 2 < G)
            def _():
                pltpu.async_copy(h_hbm.at[src_v.at[j + 2]], buf0, sem0)

            pltpu.make_async_copy(h_hbm.at[pl.ds(0, CH)], buf1, sem1).wait()
            pltpu.sync_copy(buf1, p_sh.at[dst_v.at[j + 1]], add=True)

            @pl.when(j + 3 < G)
            def _():
                pltpu.async_copy(h_hbm.at[src_v.at[j + 3]], buf1, sem1)

    plsc.subcore_barrier()
    pltpu.sync_copy(p_sh.at[pl.ds(s * RPS, RPS)],
                    p_out.at[c, pl.ds(s * RPS, RPS)])


@functools.cache
def _sc_agg():
    return pl.kernel(
        _agg_body,
        out_type=jax.ShapeDtypeStruct((NCORE, NP, H), jnp.float32),
        mesh=_mesh(),
        scratch_types=[
            pltpu.VMEM((G, CH), jnp.int32),
            pltpu.VMEM((G, CH), jnp.int32),
            pltpu.VMEM((CH, H), jnp.float32),
            pltpu.VMEM((CH, H), jnp.float32),
            pltpu.VMEM_SHARED((NP, H), jnp.float32),
            pltpu.SemaphoreType.DMA,
            pltpu.SemaphoreType.DMA,
        ],
    )


def _layer_body(p_ref, et_ref, deg_ref, h_ref, basis_ref, comp_ref,
                root_ref, cb_ref, g_ref, b_ref, o_ref, w_s):
    i = pl.program_id(0)

    @pl.when(i == 0)
    def _():
        w_s[...] = jnp.sum(comp_ref[...][:, :, None] * basis_ref[...], axis=0)

    r = 1.0 / jnp.maximum(deg_ref[...], 1.0)
    ps = p_ref[0] + p_ref[1]
    agg = jnp.dot(ps, w_s[...], preferred_element_type=jnp.float32) * r
    et = 0.1 * et_ref[...] * r
    h = h_ref[...]
    z = agg + jnp.dot(h, root_ref[...],
                      preferred_element_type=jnp.float32) + cb_ref[...] + et
    m = jnp.mean(z, axis=-1, keepdims=True)
    v = jnp.mean((z - m) ** 2, axis=-1, keepdims=True)
    z = (z - m) / jnp.sqrt(v + 1e-5) * g_ref[...] + b_ref[...]
    o_ref[...] = jnp.maximum(z, 0.0) + h


_layer = pl.pallas_call(
    _layer_body,
    grid=(TGRID,),
    in_specs=[
        pl.BlockSpec((NCORE, RB, H), lambda i: (0, i, 0)),
        pl.BlockSpec((RB, H), lambda i: (i, 0)),
        pl.BlockSpec((RB, 1), lambda i: (i, 0)),
        pl.BlockSpec((RB, H), lambda i: (i, 0)),
        pl.BlockSpec((NB, H, H), lambda i: (0, 0, 0)),
        pl.BlockSpec((NB, 1), lambda i: (0, 0)),
        pl.BlockSpec((H, H), lambda i: (0, 0)),
        pl.BlockSpec((1, H), lambda i: (0, 0)),
        pl.BlockSpec((1, H), lambda i: (0, 0)),
        pl.BlockSpec((1, H), lambda i: (0, 0)),
    ],
    out_specs=pl.BlockSpec((RB, H), lambda i: (i, 0)),
    out_shape=jax.ShapeDtypeStruct((NP, H), jnp.float32),
    scratch_shapes=[pltpu.VMEM((H, H), jnp.float32)],
)


def kernel(x, edge_index, edge_attr, Wp, bp, lnp_g, lnp_b, emb, We, be,
           basis, comp, root, conv_bias, ln_g, ln_b):
    src = edge_index[0]
    dst = edge_index[1]
    ew = edge_attr[:, 1]
    pad_e = EP - E
    ar = jnp.arange(pad_e, dtype=jnp.int32)
    src_p = jnp.concatenate([src, (ar * 997) % N])
    dst_p = jnp.concatenate([dst, N + (ar % (NP - N))])
    ew_p = jnp.concatenate([ew, jnp.zeros((pad_e,), jnp.float32)])
    src2 = src_p.reshape(R2, CH)
    dst2 = dst_p.reshape(R2, CH)
    ew2 = ew_p.reshape(R2, CH)
    x_p = jnp.pad(x, ((0, NP - N), (0, 0)))
    zer = jnp.zeros((RPS, H), jnp.float32)
    zer_ab = jnp.zeros((SLAB,), jnp.float32)

    idxa = _rank(dst2, ew2, emb, We, be[None])
    ab = _sc_ab()(idxa, ew2, zer_ab)
    h = _prep(x_p, Wp, bp[None], lnp_g[None], lnp_b[None])
    et, deg = _etdeg(ab[0].reshape(NP, RK), ab[1].reshape(NP, RK),
                     emb, We, be[None])
    for l in range(L):
        p = _sc_agg()(h, src2, dst2, zer)
        h = _layer(p, et, deg, h, basis[l], comp[l].reshape(NB, 1),
                   root[l], conv_bias[l][None], ln_g[l][None], ln_b[l][None])
    return h[:N]


# speedup vs baseline: 1.5498x; 1.0046x over previous
import functools

import jax
import jax.numpy as jnp
from jax import lax
from jax.experimental import pallas as pl
from jax.experimental.pallas import tpu as pltpu
from jax.experimental.pallas import tpu_sc as plsc

N = 10000
E = 320000
D = 128
H = 128
NB = 16
ETE = 16
L = 4

NCORE = 2
NSUB = 16
NW = NCORE * NSUB
CH = 128
K = 80
G = 40
EP = NW * K * CH
R2 = EP // CH
NP = 10240
RPS = NP // NSUB

RB = 512
TGRID = NP // RB
EB = 128

RK = 136
ABW = NP * RK
SLAB = ABW // NSUB
RPT = R2 // NSUB
BIG = 1e30


def _prep_body(x_ref, wp_ref, bp_ref, g_ref, b_ref, o_ref):
    z = jnp.dot(x_ref[...], wp_ref[...], preferred_element_type=jnp.float32)
    z = z + bp_ref[...]
    m = jnp.mean(z, axis=-1, keepdims=True)
    v = jnp.mean((z - m) ** 2, axis=-1, keepdims=True)
    z = (z - m) / jnp.sqrt(v + 1e-5) * g_ref[...] + b_ref[...]
    o_ref[...] = jnp.maximum(z, 0.0)


_prep = pl.pallas_call(
    _prep_body,
    grid=(TGRID,),
    in_specs=[
        pl.BlockSpec((RB, D), lambda i: (i, 0)),
        pl.BlockSpec((D, H), lambda i: (0, 0)),
        pl.BlockSpec((1, H), lambda i: (0, 0)),
        pl.BlockSpec((1, H), lambda i: (0, 0)),
        pl.BlockSpec((1, H), lambda i: (0, 0)),
    ],
    out_specs=pl.BlockSpec((RB, H), lambda i: (i, 0)),
    out_shape=jax.ShapeDtypeStruct((NP, H), jnp.float32),
)


def _cw_theta(emb_ref, we_ref, be_ref):
    c = jnp.dot(emb_ref[...], we_ref[:ETE, :],
                preferred_element_type=jnp.float32) + be_ref[...]
    w = we_ref[ETE, :][None, :]
    wz = w == 0.0
    th = -c / jnp.where(wz, 1.0, w)
    th = jnp.where(wz, BIG, th)
    th = jnp.clip(th, -BIG, BIG)
    return c, w, th


def _rank_body(dst_ref, ew_ref, emb_ref, we_ref, be_ref, o_ref, th_s):
    i = pl.program_id(0)

    @pl.when(i == 0)
    def _():
        _, _, th = _cw_theta(emb_ref, we_ref, be_ref)
        thb = jnp.broadcast_to(th, (H, H))
        eye = (lax.broadcasted_iota(jnp.int32, (H, H), 0)
               == lax.broadcasted_iota(jnp.int32, (H, H), 1))
        th_s[...] = jnp.sum(jnp.where(eye, thb, 0.0), axis=-1, keepdims=True)

    t = ew_ref[...]
    gt = (t[:, None, :] > th_s[...][None, :, :]).astype(jnp.float32)
    r = jnp.sum(gt, axis=1).astype(jnp.int32)
    o_ref[...] = dst_ref[...] * RK + r


_rank = pl.pallas_call(
    _rank_body,
    grid=(R2 // EB,),
    in_specs=[
        pl.BlockSpec((EB, CH), lambda i: (i, 0)),
        pl.BlockSpec((EB, CH), lambda i: (i, 0)),
        pl.BlockSpec((1, ETE), lambda i: (0, 0)),
        pl.BlockSpec((ETE + 1, H), lambda i: (0, 0)),
        pl.BlockSpec((1, H), lambda i: (0, 0)),
    ],
    out_specs=pl.BlockSpec((EB, CH), lambda i: (i, 0)),
    out_shape=jax.ShapeDtypeStruct((R2, CH), jnp.int32),
    scratch_shapes=[pltpu.VMEM((H, 1), jnp.float32)],
)


def _etdeg_body(a_ref, b_ref, emb_ref, we_ref, be_ref, et_ref, deg_ref,
                tc_s, tw_s):
    i = pl.program_id(0)

    @pl.when(i == 0)
    def _():
        c, w, th = _cw_theta(emb_ref, we_ref, be_ref)
        thv = th[0]
        lt = (thv[:, None] < thv[None, :]).astype(jnp.float32)
        ik = lax.broadcasted_iota(jnp.int32, (H, H), 0)
        ih = lax.broadcasted_iota(jnp.int32, (H, H), 1)
        tie = ((thv[:, None] == thv[None, :]) & (ik < ih)).astype(jnp.float32)
        pos = jnp.sum(lt + tie, axis=0)
        rr = lax.broadcasted_iota(jnp.int32, (RK, H), 0)
        rr = rr.astype(jnp.float32)
        posb = pos[None, :]
        a1 = (rr > posb).astype(jnp.float32)
        m = ((w > 0.0).astype(jnp.float32) * a1
             + (w < 0.0).astype(jnp.float32) * (1.0 - a1)
             + (w == 0.0).astype(jnp.float32)
             * (c > 0.0).astype(jnp.float32))
        tc_s[...] = c * m
        tw_s[...] = w * m

    et_ref[...] = (
        jnp.dot(a_ref[...], tc_s[...], preferred_element_type=jnp.float32)
        + jnp.dot(b_ref[...], tw_s[...], preferred_element_type=jnp.float32))
    deg_ref[...] = jnp.sum(a_ref[...], axis=-1, keepdims=True)


_etdeg = pl.pallas_call(
    _etdeg_body,
    grid=(TGRID,),
    in_specs=[
        pl.BlockSpec((RB, RK), lambda i: (i, 0)),
        pl.BlockSpec((RB, RK), lambda i: (i, 0)),
        pl.BlockSpec((1, ETE), lambda i: (0, 0)),
        pl.BlockSpec((ETE + 1, H), lambda i: (0, 0)),
        pl.BlockSpec((1, H), lambda i: (0, 0)),
    ],
    out_specs=(
        pl.BlockSpec((RB, H), lambda i: (i, 0)),
        pl.BlockSpec((RB, 1), lambda i: (i, 0)),
    ),
    out_shape=(
        jax.ShapeDtypeStruct((NP, H), jnp.float32),
        jax.ShapeDtypeStruct((NP, 1), jnp.float32),
    ),
    scratch_shapes=[
        pltpu.VMEM((RK, H), jnp.float32),
        pltpu.VMEM((RK, H), jnp.float32),
    ],
)


@functools.cache
def _mesh():
    return plsc.VectorSubcoreMesh(
        core_axis_name="c", subcore_axis_name="s",
        num_cores=NCORE, num_subcores=NSUB)


def _ab_body(idx_hbm, ew_hbm, zer_hbm, ab_out,
             idx_v, val_v, ones_v, acc_sh, sem):
    c = lax.axis_index("c")
    s = lax.axis_index("s")
    pltpu.async_copy(zer_hbm, acc_sh.at[pl.ds(s * SLAB, SLAB)], sem)
    pltpu.sync_copy(idx_hbm.at[pl.ds(s * RPT, RPT)], idx_v)

    @pl.when(c == 1)
    def _():
        pltpu.sync_copy(ew_hbm.at[pl.ds(s * RPT, RPT)], val_v)

    @pl.when(c == 0)
    def _():
        for i in range(CH // 16):
            ones_v[pl.ds(i * 16, 16)] = jnp.full((16,), 1.0, jnp.float32)

    pltpu.make_async_copy(zer_hbm, acc_sh.at[pl.ds(s * SLAB, SLAB)],
                          sem).wait()
    plsc.subcore_barrier()

    FD = 8

    @pl.loop(0, RPT, step=FD)
    def _chunks(j):
        for q in range(FD):
            @pl.when(c == 0)
            def _():
                pltpu.async_copy(ones_v, acc_sh.at[idx_v.at[j + q]], sem,
                                 add=True)

            @pl.when(c == 1)
            def _():
                pltpu.async_copy(val_v.at[j + q], acc_sh.at[idx_v.at[j + q]],
                                 sem, add=True)
        for q in range(FD):
            pltpu.make_async_copy(
                ones_v, acc_sh.at[idx_v.at[j + q]], sem).wait()

    plsc.subcore_barrier()
    pltpu.sync_copy(acc_sh.at[pl.ds(s * SLAB, SLAB)],
                    ab_out.at[c, pl.ds(s * SLAB, SLAB)])


@functools.cache
def _sc_ab():
    return pl.kernel(
        _ab_body,
        out_type=jax.ShapeDtypeStruct((NCORE, ABW), jnp.float32),
        mesh=_mesh(),
        scratch_types=[
            pltpu.VMEM((RPT, CH), jnp.int32),
            pltpu.VMEM((RPT, CH), jnp.float32),
            pltpu.VMEM((CH,), jnp.float32),
            pltpu.VMEM_SHARED((ABW,), jnp.float32),
            pltpu.SemaphoreType.DMA,
        ],
    )


def _agg_body(h_hbm, src_hbm, dst_hbm, zer_hbm, p_out,
              src_v, dst_v, buf0, buf1, p_sh, sem0, sem1):
    c = lax.axis_index("c")
    s = lax.axis_index("s")
    w = c * NSUB + s
    pltpu.async_copy(zer_hbm, p_sh.at[pl.ds(s * RPS, RPS)], sem0)
    pltpu.sync_copy(src_hbm.at[pl.ds(w * K, G)], src_v)
    pltpu.sync_copy(dst_hbm.at[pl.ds(w * K, G)], dst_v)
    pltpu.make_async_copy(zer_hbm, p_sh.at[pl.ds(s * RPS, RPS)], sem0).wait()
    plsc.subcore_barrier()
    for half in range(K // G):
        if half > 0:
            pltpu.sync_copy(src_hbm.at[pl.ds(w * K + half * G, G)], src_v)
            pltpu.sync_copy(dst_hbm.at[pl.ds(w * K + half * G, G)], dst_v)
        pltpu.async_copy(h_hbm.at[src_v.at[0]], buf0, sem0)
        pltpu.async_copy(h_hbm.at[src_v.at[1]], buf1, sem1)

        @pl.loop(0, G, step=2)
        def _chunks(j):
            pltpu.make_async_copy(h_hbm.at[pl.ds(0, CH)], buf0, sem0).wait()
            pltpu.sync_copy(buf0, p_sh.at[dst_v.at[j]], add=True)

            @pl.when(j + 2 < G)
            def _():
                pltpu.async_copy(h_hbm.at[src_v.at[j + 2]], buf0, sem0)

            pltpu.make_async_copy(h_hbm.at[pl.ds(0, CH)], buf1, sem1).wait()
            pltpu.sync_copy(buf1, p_sh.at[dst_v.at[j + 1]], add=True)

            @pl.when(j + 3 < G)
            def _():
                pltpu.async_copy(h_hbm.at[src_v.at[j + 3]], buf1, sem1)

    plsc.subcore_barrier()
    pltpu.sync_copy(p_sh.at[pl.ds(s * RPS, RPS)],
                    p_out.at[c, pl.ds(s * RPS, RPS)])


@functools.cache
def _sc_agg():
    return pl.kernel(
        _agg_body,
        out_type=jax.ShapeDtypeStruct((NCORE, NP, H), jnp.float32),
        mesh=_mesh(),
        scratch_types=[
            pltpu.VMEM((G, CH), jnp.int32),
            pltpu.VMEM((G, CH), jnp.int32),
            pltpu.VMEM((CH, H), jnp.float32),
            pltpu.VMEM((CH, H), jnp.float32),
            pltpu.VMEM_SHARED((NP, H), jnp.float32),
            pltpu.SemaphoreType.DMA,
            pltpu.SemaphoreType.DMA,
        ],
    )


def _layer_body(p_ref, et_ref, deg_ref, h_ref, basis_ref, comp_ref,
                root_ref, cb_ref, g_ref, b_ref, o_ref, w_s):
    i = pl.program_id(0)

    @pl.when(i == 0)
    def _():
        w_s[...] = jnp.sum(comp_ref[...][:, :, None] * basis_ref[...], axis=0)

    r = 1.0 / jnp.maximum(deg_ref[...], 1.0)
    ps = p_ref[0] + p_ref[1]
    agg = jnp.dot(ps, w_s[...], preferred_element_type=jnp.float32) * r
    et = 0.1 * et_ref[...] * r
    h = h_ref[...]
    z = agg + jnp.dot(h, root_ref[...],
                      preferred_element_type=jnp.float32) + cb_ref[...] + et
    m = jnp.mean(z, axis=-1, keepdims=True)
    v = jnp.mean((z - m) ** 2, axis=-1, keepdims=True)
    z = (z - m) / jnp.sqrt(v + 1e-5) * g_ref[...] + b_ref[...]
    o_ref[...] = jnp.maximum(z, 0.0) + h


_layer = pl.pallas_call(
    _layer_body,
    grid=(TGRID,),
    in_specs=[
        pl.BlockSpec((NCORE, RB, H), lambda i: (0, i, 0)),
        pl.BlockSpec((RB, H), lambda i: (i, 0)),
        pl.BlockSpec((RB, 1), lambda i: (i, 0)),
        pl.BlockSpec((RB, H), lambda i: (i, 0)),
        pl.BlockSpec((NB, H, H), lambda i: (0, 0, 0)),
        pl.BlockSpec((NB, 1), lambda i: (0, 0)),
        pl.BlockSpec((H, H), lambda i: (0, 0)),
        pl.BlockSpec((1, H), lambda i: (0, 0)),
        pl.BlockSpec((1, H), lambda i: (0, 0)),
        pl.BlockSpec((1, H), lambda i: (0, 0)),
    ],
    out_specs=pl.BlockSpec((RB, H), lambda i: (i, 0)),
    out_shape=jax.ShapeDtypeStruct((NP, H), jnp.float32),
    scratch_shapes=[pltpu.VMEM((H, H), jnp.float32)],
)


def kernel(x, edge_index, edge_attr, Wp, bp, lnp_g, lnp_b, emb, We, be,
           basis, comp, root, conv_bias, ln_g, ln_b):
    src = edge_index[0]
    dst = edge_index[1]
    ew = edge_attr[:, 1]
    pad_e = EP - E
    ar = jnp.arange(pad_e, dtype=jnp.int32)
    src_p = jnp.concatenate([src, (ar * 997) % N])
    dst_p = jnp.concatenate([dst, N + (ar % (NP - N))])
    ew_p = jnp.concatenate([ew, jnp.zeros((pad_e,), jnp.float32)])
    src2 = src_p.reshape(R2, CH)
    dst2 = dst_p.reshape(R2, CH)
    ew2 = ew_p.reshape(R2, CH)
    x_p = jnp.pad(x, ((0, NP - N), (0, 0)))
    zer = jnp.zeros((RPS, H), jnp.float32)
    zer_ab = jnp.zeros((SLAB,), jnp.float32)

    idxa = _rank(dst2, ew2, emb, We, be[None])
    ab = _sc_ab()(idxa, ew2, zer_ab)
    h = _prep(x_p, Wp, bp[None], lnp_g[None], lnp_b[None])
    et, deg = _etdeg(ab[0].reshape(NP, RK), ab[1].reshape(NP, RK),
                     emb, We, be[None])
    for l in range(L):
        p = _sc_agg()(h, src2, dst2, zer)
        h = _layer(p, et, deg, h, basis[l], comp[l].reshape(NB, 1),
                   root[l], conv_bias[l][None], ln_g[l][None], ln_b[l][None])
    return h[:N]


# speedup vs baseline: 1.6161x; 1.0428x over previous
import functools

import jax
import jax.numpy as jnp
from jax import lax
from jax.experimental import pallas as pl
from jax.experimental.pallas import tpu as pltpu
from jax.experimental.pallas import tpu_sc as plsc

N = 10000
E = 320000
D = 128
H = 128
NB = 16
ETE = 16
L = 4

NCORE = 2
NSUB = 16
NW = NCORE * NSUB
CH = 128
K = 80
G = 40
EP = NW * K * CH
R2 = EP // CH
NP = 10240
RPS = NP // NSUB

RB = 1024
TGRID = NP // RB
EB = 128

RK = 136
ABW = NP * RK
SLAB = ABW // NSUB
RPT = R2 // NSUB
BIG = 1e30


def _prep_body(x_ref, wp_ref, bp_ref, g_ref, b_ref, o_ref):
    z = jnp.dot(x_ref[...], wp_ref[...], preferred_element_type=jnp.float32)
    z = z + bp_ref[...]
    m = jnp.mean(z, axis=-1, keepdims=True)
    v = jnp.mean((z - m) ** 2, axis=-1, keepdims=True)
    z = (z - m) / jnp.sqrt(v + 1e-5) * g_ref[...] + b_ref[...]
    o_ref[...] = jnp.maximum(z, 0.0)


_prep = pl.pallas_call(
    _prep_body,
    grid=(TGRID,),
    in_specs=[
        pl.BlockSpec((RB, D), lambda i: (i, 0)),
        pl.BlockSpec((D, H), lambda i: (0, 0)),
        pl.BlockSpec((1, H), lambda i: (0, 0)),
        pl.BlockSpec((1, H), lambda i: (0, 0)),
        pl.BlockSpec((1, H), lambda i: (0, 0)),
    ],
    out_specs=pl.BlockSpec((RB, H), lambda i: (i, 0)),
    out_shape=jax.ShapeDtypeStruct((NP, H), jnp.float32),
)


def _cw_theta(emb_ref, we_ref, be_ref):
    c = jnp.dot(emb_ref[...], we_ref[:ETE, :],
                preferred_element_type=jnp.float32) + be_ref[...]
    w = we_ref[ETE, :][None, :]
    wz = w == 0.0
    th = -c / jnp.where(wz, 1.0, w)
    th = jnp.where(wz, BIG, th)
    th = jnp.clip(th, -BIG, BIG)
    return c, w, th


def _rank_body(dst_ref, ew_ref, emb_ref, we_ref, be_ref, o_ref, th_s):
    i = pl.program_id(0)

    @pl.when(i == 0)
    def _():
        _, _, th = _cw_theta(emb_ref, we_ref, be_ref)
        thb = jnp.broadcast_to(th, (H, H))
        eye = (lax.broadcasted_iota(jnp.int32, (H, H), 0)
               == lax.broadcasted_iota(jnp.int32, (H, H), 1))
        th_s[...] = jnp.sum(jnp.where(eye, thb, 0.0), axis=-1, keepdims=True)

    t = ew_ref[...]
    gt = (t[:, None, :] > th_s[...][None, :, :]).astype(jnp.float32)
    r = jnp.sum(gt, axis=1).astype(jnp.int32)
    o_ref[...] = dst_ref[...] * RK + r


_rank = pl.pallas_call(
    _rank_body,
    grid=(R2 // EB,),
    in_specs=[
        pl.BlockSpec((EB, CH), lambda i: (i, 0)),
        pl.BlockSpec((EB, CH), lambda i: (i, 0)),
        pl.BlockSpec((1, ETE), lambda i: (0, 0)),
        pl.BlockSpec((ETE + 1, H), lambda i: (0, 0)),
        pl.BlockSpec((1, H), lambda i: (0, 0)),
    ],
    out_specs=pl.BlockSpec((EB, CH), lambda i: (i, 0)),
    out_shape=jax.ShapeDtypeStruct((R2, CH), jnp.int32),
    scratch_shapes=[pltpu.VMEM((H, 1), jnp.float32)],
)


def _etdeg_body(a_ref, b_ref, emb_ref, we_ref, be_ref, et_ref, deg_ref,
                tc_s, tw_s):
    i = pl.program_id(0)

    @pl.when(i == 0)
    def _():
        c, w, th = _cw_theta(emb_ref, we_ref, be_ref)
        thv = th[0]
        lt = (thv[:, None] < thv[None, :]).astype(jnp.float32)
        ik = lax.broadcasted_iota(jnp.int32, (H, H), 0)
        ih = lax.broadcasted_iota(jnp.int32, (H, H), 1)
        tie = ((thv[:, None] == thv[None, :]) & (ik < ih)).astype(jnp.float32)
        pos = jnp.sum(lt + tie, axis=0)
        rr = lax.broadcasted_iota(jnp.int32, (RK, H), 0)
        rr = rr.astype(jnp.float32)
        posb = pos[None, :]
        a1 = (rr > posb).astype(jnp.float32)
        m = ((w > 0.0).astype(jnp.float32) * a1
             + (w < 0.0).astype(jnp.float32) * (1.0 - a1)
             + (w == 0.0).astype(jnp.float32)
             * (c > 0.0).astype(jnp.float32))
        tc_s[...] = c * m
        tw_s[...] = w * m

    et_ref[...] = (
        jnp.dot(a_ref[...], tc_s[...], preferred_element_type=jnp.float32)
        + jnp.dot(b_ref[...], tw_s[...], preferred_element_type=jnp.float32))
    deg_ref[...] = jnp.sum(a_ref[...], axis=-1, keepdims=True)


_etdeg = pl.pallas_call(
    _etdeg_body,
    grid=(TGRID,),
    in_specs=[
        pl.BlockSpec((RB, RK), lambda i: (i, 0)),
        pl.BlockSpec((RB, RK), lambda i: (i, 0)),
        pl.BlockSpec((1, ETE), lambda i: (0, 0)),
        pl.BlockSpec((ETE + 1, H), lambda i: (0, 0)),
        pl.BlockSpec((1, H), lambda i: (0, 0)),
    ],
    out_specs=(
        pl.BlockSpec((RB, H), lambda i: (i, 0)),
        pl.BlockSpec((RB, 1), lambda i: (i, 0)),
    ),
    out_shape=(
        jax.ShapeDtypeStruct((NP, H), jnp.float32),
        jax.ShapeDtypeStruct((NP, 1), jnp.float32),
    ),
    scratch_shapes=[
        pltpu.VMEM((RK, H), jnp.float32),
        pltpu.VMEM((RK, H), jnp.float32),
    ],
)


@functools.cache
def _mesh():
    return plsc.VectorSubcoreMesh(
        core_axis_name="c", subcore_axis_name="s",
        num_cores=NCORE, num_subcores=NSUB)


def _ab_body(idx_hbm, ew_hbm, zer_hbm, ab_out,
             idx_v, val_v, ones_v, acc_sh, sem):
    c = lax.axis_index("c")
    s = lax.axis_index("s")
    pltpu.async_copy(zer_hbm, acc_sh.at[pl.ds(s * SLAB, SLAB)], sem)
    pltpu.sync_copy(idx_hbm.at[pl.ds(s * RPT, RPT)], idx_v)

    @pl.when(c == 1)
    def _():
        pltpu.sync_copy(ew_hbm.at[pl.ds(s * RPT, RPT)], val_v)

    @pl.when(c == 0)
    def _():
        for i in range(CH // 16):
            ones_v[pl.ds(i * 16, 16)] = jnp.full((16,), 1.0, jnp.float32)

    pltpu.make_async_copy(zer_hbm, acc_sh.at[pl.ds(s * SLAB, SLAB)],
                          sem).wait()
    plsc.subcore_barrier()

    FD = 8

    @pl.loop(0, RPT, step=FD)
    def _chunks(j):
        for q in range(FD):
            @pl.when(c == 0)
            def _():
                pltpu.async_copy(ones_v, acc_sh.at[idx_v.at[j + q]], sem,
                                 add=True)

            @pl.when(c == 1)
            def _():
                pltpu.async_copy(val_v.at[j + q], acc_sh.at[idx_v.at[j + q]],
                                 sem, add=True)
        for q in range(FD):
            pltpu.make_async_copy(
                ones_v, acc_sh.at[idx_v.at[j + q]], sem).wait()

    plsc.subcore_barrier()
    pltpu.sync_copy(acc_sh.at[pl.ds(s * SLAB, SLAB)],
                    ab_out.at[c, pl.ds(s * SLAB, SLAB)])


@functools.cache
def _sc_ab():
    return pl.kernel(
        _ab_body,
        out_type=jax.ShapeDtypeStruct((NCORE, ABW), jnp.float32),
        mesh=_mesh(),
        scratch_types=[
            pltpu.VMEM((RPT, CH), jnp.int32),
            pltpu.VMEM((RPT, CH), jnp.float32),
            pltpu.VMEM((CH,), jnp.float32),
            pltpu.VMEM_SHARED((ABW,), jnp.float32),
            pltpu.SemaphoreType.DMA,
        ],
    )


def _agg_body(h_hbm, src_hbm, dst_hbm, zer_hbm, p_out,
              src_v, dst_v, buf0, buf1, p_sh, sem0, sem1):
    c = lax.axis_index("c")
    s = lax.axis_index("s")
    w = c * NSUB + s
    pltpu.async_copy(zer_hbm, p_sh.at[pl.ds(s * RPS, RPS)], sem0)
    pltpu.sync_copy(src_hbm.at[pl.ds(w * K, G)], src_v)
    pltpu.sync_copy(dst_hbm.at[pl.ds(w * K, G)], dst_v)
    pltpu.make_async_copy(zer_hbm, p_sh.at[pl.ds(s * RPS, RPS)], sem0).wait()
    plsc.subcore_barrier()
    for half in range(K // G):
        if half > 0:
            pltpu.sync_copy(src_hbm.at[pl.ds(w * K + half * G, G)], src_v)
            pltpu.sync_copy(dst_hbm.at[pl.ds(w * K + half * G, G)], dst_v)
        pltpu.async_copy(h_hbm.at[src_v.at[0]], buf0, sem0)
        pltpu.async_copy(h_hbm.at[src_v.at[1]], buf1, sem1)

        @pl.loop(0, G, step=2)
        def _chunks(j):
            pltpu.make_async_copy(h_hbm.at[pl.ds(0, CH)], buf0, sem0).wait()
            pltpu.sync_copy(buf0, p_sh.at[dst_v.at[j]], add=True)

            @pl.when(j + 2 < G)
            def _():
                pltpu.async_copy(h_hbm.at[src_v.at[j + 2]], buf0, sem0)

            pltpu.make_async_copy(h_hbm.at[pl.ds(0, CH)], buf1, sem1).wait()
            pltpu.sync_copy(buf1, p_sh.at[dst_v.at[j + 1]], add=True)

            @pl.when(j + 3 < G)
            def _():
                pltpu.async_copy(h_hbm.at[src_v.at[j + 3]], buf1, sem1)

    plsc.subcore_barrier()
    pltpu.sync_copy(p_sh.at[pl.ds(s * RPS, RPS)],
                    p_out.at[c, pl.ds(s * RPS, RPS)])


@functools.cache
def _sc_agg():
    return pl.kernel(
        _agg_body,
        out_type=jax.ShapeDtypeStruct((NCORE, NP, H), jnp.float32),
        mesh=_mesh(),
        scratch_types=[
            pltpu.VMEM((G, CH), jnp.int32),
            pltpu.VMEM((G, CH), jnp.int32),
            pltpu.VMEM((CH, H), jnp.float32),
            pltpu.VMEM((CH, H), jnp.float32),
            pltpu.VMEM_SHARED((NP, H), jnp.float32),
            pltpu.SemaphoreType.DMA,
            pltpu.SemaphoreType.DMA,
        ],
    )


def _layer_body(p_ref, et_ref, deg_ref, h_ref, basis_ref, comp_ref,
                root_ref, cb_ref, g_ref, b_ref, o_ref, w_s):
    i = pl.program_id(0)

    @pl.when(i == 0)
    def _():
        w_s[...] = jnp.sum(comp_ref[...][:, :, None] * basis_ref[...], axis=0)

    r = 1.0 / jnp.maximum(deg_ref[...], 1.0)
    ps = p_ref[0] + p_ref[1]
    agg = jnp.dot(ps, w_s[...], preferred_element_type=jnp.float32) * r
    et = 0.1 * et_ref[...] * r
    h = h_ref[...]
    z = agg + jnp.dot(h, root_ref[...],
                      preferred_element_type=jnp.float32) + cb_ref[...] + et
    m = jnp.mean(z, axis=-1, keepdims=True)
    v = jnp.mean((z - m) ** 2, axis=-1, keepdims=True)
    z = (z - m) / jnp.sqrt(v + 1e-5) * g_ref[...] + b_ref[...]
    o_ref[...] = jnp.maximum(z, 0.0) + h


_layer = pl.pallas_call(
    _layer_body,
    grid=(TGRID,),
    in_specs=[
        pl.BlockSpec((NCORE, RB, H), lambda i: (0, i, 0)),
        pl.BlockSpec((RB, H), lambda i: (i, 0)),
        pl.BlockSpec((RB, 1), lambda i: (i, 0)),
        pl.BlockSpec((RB, H), lambda i: (i, 0)),
        pl.BlockSpec((NB, H, H), lambda i: (0, 0, 0)),
        pl.BlockSpec((NB, 1), lambda i: (0, 0)),
        pl.BlockSpec((H, H), lambda i: (0, 0)),
        pl.BlockSpec((1, H), lambda i: (0, 0)),
        pl.BlockSpec((1, H), lambda i: (0, 0)),
        pl.BlockSpec((1, H), lambda i: (0, 0)),
    ],
    out_specs=pl.BlockSpec((RB, H), lambda i: (i, 0)),
    out_shape=jax.ShapeDtypeStruct((NP, H), jnp.float32),
    scratch_shapes=[pltpu.VMEM((H, H), jnp.float32)],
)


def kernel(x, edge_index, edge_attr, Wp, bp, lnp_g, lnp_b, emb, We, be,
           basis, comp, root, conv_bias, ln_g, ln_b):
    src = edge_index[0]
    dst = edge_index[1]
    ew = edge_attr[:, 1]
    pad_e = EP - E
    ar = jnp.arange(pad_e, dtype=jnp.int32)
    src_p = jnp.concatenate([src, (ar * 997) % N])
    dst_p = jnp.concatenate([dst, N + (ar % (NP - N))])
    ew_p = jnp.concatenate([ew, jnp.zeros((pad_e,), jnp.float32)])
    src2 = src_p.reshape(R2, CH)
    dst2 = dst_p.reshape(R2, CH)
    ew2 = ew_p.reshape(R2, CH)
    x_p = jnp.pad(x, ((0, NP - N), (0, 0)))
    zer = jnp.zeros((RPS, H), jnp.float32)
    zer_ab = jnp.zeros((SLAB,), jnp.float32)

    idxa = _rank(dst2, ew2, emb, We, be[None])
    ab = _sc_ab()(idxa, ew2, zer_ab)
    h = _prep(x_p, Wp, bp[None], lnp_g[None], lnp_b[None])
    et, deg = _etdeg(ab[0].reshape(NP, RK), ab[1].reshape(NP, RK),
                     emb, We, be[None])
    for l in range(L):
        p = _sc_agg()(h, src2, dst2, zer)
        h = _layer(p, et, deg, h, basis[l], comp[l].reshape(NB, 1),
                   root[l], conv_bias[l][None], ln_g[l][None], ln_b[l][None])
    return h[:N]


# speedup vs baseline: 1.6378x; 1.0134x over previous
import functools

import jax
import jax.numpy as jnp
from jax import lax
from jax.experimental import pallas as pl
from jax.experimental.pallas import tpu as pltpu
from jax.experimental.pallas import tpu_sc as plsc

N = 10000
E = 320000
D = 128
H = 128
NB = 16
ETE = 16
L = 4

NCORE = 2
NSUB = 16
NW = NCORE * NSUB
CH = 128
K = 80
G = 40
EP = NW * K * CH
R2 = EP // CH
NP = 10240
RPS = NP // NSUB

RB = 2048
TGRID = NP // RB
EB = 128

RK = 136
ABW = NP * RK
SLAB = ABW // NSUB
RPT = R2 // NSUB
BIG = 1e30


def _prep_body(x_ref, wp_ref, bp_ref, g_ref, b_ref, o_ref):
    z = jnp.dot(x_ref[...], wp_ref[...], preferred_element_type=jnp.float32)
    z = z + bp_ref[...]
    m = jnp.mean(z, axis=-1, keepdims=True)
    v = jnp.mean((z - m) ** 2, axis=-1, keepdims=True)
    z = (z - m) / jnp.sqrt(v + 1e-5) * g_ref[...] + b_ref[...]
    o_ref[...] = jnp.maximum(z, 0.0)


_prep = pl.pallas_call(
    _prep_body,
    grid=(TGRID,),
    in_specs=[
        pl.BlockSpec((RB, D), lambda i: (i, 0)),
        pl.BlockSpec((D, H), lambda i: (0, 0)),
        pl.BlockSpec((1, H), lambda i: (0, 0)),
        pl.BlockSpec((1, H), lambda i: (0, 0)),
        pl.BlockSpec((1, H), lambda i: (0, 0)),
    ],
    out_specs=pl.BlockSpec((RB, H), lambda i: (i, 0)),
    out_shape=jax.ShapeDtypeStruct((NP, H), jnp.float32),
)


def _cw_theta(emb_ref, we_ref, be_ref):
    c = jnp.dot(emb_ref[...], we_ref[:ETE, :],
                preferred_element_type=jnp.float32) + be_ref[...]
    w = we_ref[ETE, :][None, :]
    wz = w == 0.0
    th = -c / jnp.where(wz, 1.0, w)
    th = jnp.where(wz, BIG, th)
    th = jnp.clip(th, -BIG, BIG)
    return c, w, th


def _rank_body(dst_ref, ew_ref, emb_ref, we_ref, be_ref, o_ref, th_s):
    i = pl.program_id(0)

    @pl.when(i == 0)
    def _():
        _, _, th = _cw_theta(emb_ref, we_ref, be_ref)
        thb = jnp.broadcast_to(th, (H, H))
        eye = (lax.broadcasted_iota(jnp.int32, (H, H), 0)
               == lax.broadcasted_iota(jnp.int32, (H, H), 1))
        th_s[...] = jnp.sum(jnp.where(eye, thb, 0.0), axis=-1, keepdims=True)

    t = ew_ref[...]
    gt = (t[:, None, :] > th_s[...][None, :, :]).astype(jnp.float32)
    r = jnp.sum(gt, axis=1).astype(jnp.int32)
    o_ref[...] = dst_ref[...] * RK + r


_rank = pl.pallas_call(
    _rank_body,
    grid=(R2 // EB,),
    in_specs=[
        pl.BlockSpec((EB, CH), lambda i: (i, 0)),
        pl.BlockSpec((EB, CH), lambda i: (i, 0)),
        pl.BlockSpec((1, ETE), lambda i: (0, 0)),
        pl.BlockSpec((ETE + 1, H), lambda i: (0, 0)),
        pl.BlockSpec((1, H), lambda i: (0, 0)),
    ],
    out_specs=pl.BlockSpec((EB, CH), lambda i: (i, 0)),
    out_shape=jax.ShapeDtypeStruct((R2, CH), jnp.int32),
    scratch_shapes=[pltpu.VMEM((H, 1), jnp.float32)],
)


def _etdeg_body(a_ref, b_ref, emb_ref, we_ref, be_ref, et_ref, deg_ref,
                tc_s, tw_s):
    i = pl.program_id(0)

    @pl.when(i == 0)
    def _():
        c, w, th = _cw_theta(emb_ref, we_ref, be_ref)
        thv = th[0]
        lt = (thv[:, None] < thv[None, :]).astype(jnp.float32)
        ik = lax.broadcasted_iota(jnp.int32, (H, H), 0)
        ih = lax.broadcasted_iota(jnp.int32, (H, H), 1)
        tie = ((thv[:, None] == thv[None, :]) & (ik < ih)).astype(jnp.float32)
        pos = jnp.sum(lt + tie, axis=0)
        rr = lax.broadcasted_iota(jnp.int32, (RK, H), 0)
        rr = rr.astype(jnp.float32)
        posb = pos[None, :]
        a1 = (rr > posb).astype(jnp.float32)
        m = ((w > 0.0).astype(jnp.float32) * a1
             + (w < 0.0).astype(jnp.float32) * (1.0 - a1)
             + (w == 0.0).astype(jnp.float32)
             * (c > 0.0).astype(jnp.float32))
        tc_s[...] = c * m
        tw_s[...] = w * m

    et_ref[...] = (
        jnp.dot(a_ref[...], tc_s[...], preferred_element_type=jnp.float32)
        + jnp.dot(b_ref[...], tw_s[...], preferred_element_type=jnp.float32))
    deg_ref[...] = jnp.sum(a_ref[...], axis=-1, keepdims=True)


_etdeg = pl.pallas_call(
    _etdeg_body,
    grid=(TGRID,),
    in_specs=[
        pl.BlockSpec((RB, RK), lambda i: (i, 0)),
        pl.BlockSpec((RB, RK), lambda i: (i, 0)),
        pl.BlockSpec((1, ETE), lambda i: (0, 0)),
        pl.BlockSpec((ETE + 1, H), lambda i: (0, 0)),
        pl.BlockSpec((1, H), lambda i: (0, 0)),
    ],
    out_specs=(
        pl.BlockSpec((RB, H), lambda i: (i, 0)),
        pl.BlockSpec((RB, 1), lambda i: (i, 0)),
    ),
    out_shape=(
        jax.ShapeDtypeStruct((NP, H), jnp.float32),
        jax.ShapeDtypeStruct((NP, 1), jnp.float32),
    ),
    scratch_shapes=[
        pltpu.VMEM((RK, H), jnp.float32),
        pltpu.VMEM((RK, H), jnp.float32),
    ],
)


@functools.cache
def _mesh():
    return plsc.VectorSubcoreMesh(
        core_axis_name="c", subcore_axis_name="s",
        num_cores=NCORE, num_subcores=NSUB)


def _ab_body(idx_hbm, ew_hbm, zer_hbm, ab_out,
             idx_v, val_v, ones_v, acc_sh, sem):
    c = lax.axis_index("c")
    s = lax.axis_index("s")
    pltpu.async_copy(zer_hbm, acc_sh.at[pl.ds(s * SLAB, SLAB)], sem)
    pltpu.sync_copy(idx_hbm.at[pl.ds(s * RPT, RPT)], idx_v)

    @pl.when(c == 1)
    def _():
        pltpu.sync_copy(ew_hbm.at[pl.ds(s * RPT, RPT)], val_v)

    @pl.when(c == 0)
    def _():
        for i in range(CH // 16):
            ones_v[pl.ds(i * 16, 16)] = jnp.full((16,), 1.0, jnp.float32)

    pltpu.make_async_copy(zer_hbm, acc_sh.at[pl.ds(s * SLAB, SLAB)],
                          sem).wait()
    plsc.subcore_barrier()

    FD = 8

    @pl.loop(0, RPT, step=FD)
    def _chunks(j):
        for q in range(FD):
            @pl.when(c == 0)
            def _():
                pltpu.async_copy(ones_v, acc_sh.at[idx_v.at[j + q]], sem,
                                 add=True)

            @pl.when(c == 1)
            def _():
                pltpu.async_copy(val_v.at[j + q], acc_sh.at[idx_v.at[j + q]],
                                 sem, add=True)
        for q in range(FD):
            pltpu.make_async_copy(
                ones_v, acc_sh.at[idx_v.at[j + q]], sem).wait()

    plsc.subcore_barrier()
    pltpu.sync_copy(acc_sh.at[pl.ds(s * SLAB, SLAB)],
                    ab_out.at[c, pl.ds(s * SLAB, SLAB)])


@functools.cache
def _sc_ab():
    return pl.kernel(
        _ab_body,
        out_type=jax.ShapeDtypeStruct((NCORE, ABW), jnp.float32),
        mesh=_mesh(),
        scratch_types=[
            pltpu.VMEM((RPT, CH), jnp.int32),
            pltpu.VMEM((RPT, CH), jnp.float32),
            pltpu.VMEM((CH,), jnp.float32),
            pltpu.VMEM_SHARED((ABW,), jnp.float32),
            pltpu.SemaphoreType.DMA,
        ],
    )


def _agg_body(h_hbm, src_hbm, dst_hbm, zer_hbm, p_out,
              src_v, dst_v, buf0, buf1, p_sh, sem0, sem1):
    c = lax.axis_index("c")
    s = lax.axis_index("s")
    w = c * NSUB + s
    pltpu.async_copy(zer_hbm, p_sh.at[pl.ds(s * RPS, RPS)], sem0)
    pltpu.sync_copy(src_hbm.at[pl.ds(w * K, G)], src_v)
    pltpu.sync_copy(dst_hbm.at[pl.ds(w * K, G)], dst_v)
    pltpu.make_async_copy(zer_hbm, p_sh.at[pl.ds(s * RPS, RPS)], sem0).wait()
    plsc.subcore_barrier()
    for half in range(K // G):
        if half > 0:
            pltpu.sync_copy(src_hbm.at[pl.ds(w * K + half * G, G)], src_v)
            pltpu.sync_copy(dst_hbm.at[pl.ds(w * K + half * G, G)], dst_v)
        pltpu.async_copy(h_hbm.at[src_v.at[0]], buf0, sem0)
        pltpu.async_copy(h_hbm.at[src_v.at[1]], buf1, sem1)

        @pl.loop(0, G, step=2)
        def _chunks(j):
            pltpu.make_async_copy(h_hbm.at[pl.ds(0, CH)], buf0, sem0).wait()
            pltpu.sync_copy(buf0, p_sh.at[dst_v.at[j]], add=True)

            @pl.when(j + 2 < G)
            def _():
                pltpu.async_copy(h_hbm.at[src_v.at[j + 2]], buf0, sem0)

            pltpu.make_async_copy(h_hbm.at[pl.ds(0, CH)], buf1, sem1).wait()
            pltpu.sync_copy(buf1, p_sh.at[dst_v.at[j + 1]], add=True)

            @pl.when(j + 3 < G)
            def _():
                pltpu.async_copy(h_hbm.at[src_v.at[j + 3]], buf1, sem1)

    plsc.subcore_barrier()
    pltpu.sync_copy(p_sh.at[pl.ds(s * RPS, RPS)],
                    p_out.at[c, pl.ds(s * RPS, RPS)])


@functools.cache
def _sc_agg():
    return pl.kernel(
        _agg_body,
        out_type=jax.ShapeDtypeStruct((NCORE, NP, H), jnp.float32),
        mesh=_mesh(),
        scratch_types=[
            pltpu.VMEM((G, CH), jnp.int32),
            pltpu.VMEM((G, CH), jnp.int32),
            pltpu.VMEM((CH, H), jnp.float32),
            pltpu.VMEM((CH, H), jnp.float32),
            pltpu.VMEM_SHARED((NP, H), jnp.float32),
            pltpu.SemaphoreType.DMA,
            pltpu.SemaphoreType.DMA,
        ],
    )


def _layer_body(p_ref, et_ref, deg_ref, h_ref, basis_ref, comp_ref,
                root_ref, cb_ref, g_ref, b_ref, o_ref, w_s):
    i = pl.program_id(0)

    @pl.when(i == 0)
    def _():
        w_s[...] = jnp.sum(comp_ref[...][:, :, None] * basis_ref[...], axis=0)

    r = 1.0 / jnp.maximum(deg_ref[...], 1.0)
    ps = p_ref[0] + p_ref[1]
    agg = jnp.dot(ps, w_s[...], preferred_element_type=jnp.float32) * r
    et = 0.1 * et_ref[...] * r
    h = h_ref[...]
    z = agg + jnp.dot(h, root_ref[...],
                      preferred_element_type=jnp.float32) + cb_ref[...] + et
    m = jnp.mean(z, axis=-1, keepdims=True)
    v = jnp.mean((z - m) ** 2, axis=-1, keepdims=True)
    z = (z - m) / jnp.sqrt(v + 1e-5) * g_ref[...] + b_ref[...]
    o_ref[...] = jnp.maximum(z, 0.0) + h


_layer = pl.pallas_call(
    _layer_body,
    grid=(TGRID,),
    in_specs=[
        pl.BlockSpec((NCORE, RB, H), lambda i: (0, i, 0)),
        pl.BlockSpec((RB, H), lambda i: (i, 0)),
        pl.BlockSpec((RB, 1), lambda i: (i, 0)),
        pl.BlockSpec((RB, H), lambda i: (i, 0)),
        pl.BlockSpec((NB, H, H), lambda i: (0, 0, 0)),
        pl.BlockSpec((NB, 1), lambda i: (0, 0)),
        pl.BlockSpec((H, H), lambda i: (0, 0)),
        pl.BlockSpec((1, H), lambda i: (0, 0)),
        pl.BlockSpec((1, H), lambda i: (0, 0)),
        pl.BlockSpec((1, H), lambda i: (0, 0)),
    ],
    out_specs=pl.BlockSpec((RB, H), lambda i: (i, 0)),
    out_shape=jax.ShapeDtypeStruct((NP, H), jnp.float32),
    scratch_shapes=[pltpu.VMEM((H, H), jnp.float32)],
)


def kernel(x, edge_index, edge_attr, Wp, bp, lnp_g, lnp_b, emb, We, be,
           basis, comp, root, conv_bias, ln_g, ln_b):
    src = edge_index[0]
    dst = edge_index[1]
    ew = edge_attr[:, 1]
    pad_e = EP - E
    ar = jnp.arange(pad_e, dtype=jnp.int32)
    src_p = jnp.concatenate([src, (ar * 997) % N])
    dst_p = jnp.concatenate([dst, N + (ar % (NP - N))])
    ew_p = jnp.concatenate([ew, jnp.zeros((pad_e,), jnp.float32)])
    src2 = src_p.reshape(R2, CH)
    dst2 = dst_p.reshape(R2, CH)
    ew2 = ew_p.reshape(R2, CH)
    x_p = jnp.pad(x, ((0, NP - N), (0, 0)))
    zer = jnp.zeros((RPS, H), jnp.float32)
    zer_ab = jnp.zeros((SLAB,), jnp.float32)

    idxa = _rank(dst2, ew2, emb, We, be[None])
    ab = _sc_ab()(idxa, ew2, zer_ab)
    h = _prep(x_p, Wp, bp[None], lnp_g[None], lnp_b[None])
    et, deg = _etdeg(ab[0].reshape(NP, RK), ab[1].reshape(NP, RK),
                     emb, We, be[None])
    for l in range(L):
        p = _sc_agg()(h, src2, dst2, zer)
        h = _layer(p, et, deg, h, basis[l], comp[l].reshape(NB, 1),
                   root[l], conv_bias[l][None], ln_g[l][None], ln_b[l][None])
    return h[:N]
